# Initial kernel scaffold; baseline (speedup 1.0000x reference)
import jax
import jax.numpy as jnp
from jax.experimental import pallas as pl


def kernel(x, pos, edge_index, W_in, b_in, W_lin, W_src, W_dst, Wp1, bp1, Wp2, bp2, Wa1, ba1, Wa2, bb2, W_out, b_out):
    raise NotImplementedError("write your pallas kernel here")


# speedup vs baseline: 6.4725x; 6.4725x over previous
import functools

import jax
import jax.numpy as jnp
from jax import lax
from jax.experimental import pallas as pl
from jax.experimental.pallas import tpu as pltpu
from jax.experimental.pallas import tpu_sc as plsc

N = 10000
E = 320000
D = 128
AH = 64
PW = 16
DS = 256
DD = 128

NC = 2
NS = 16
NW = NC * NS

CG = 80
CS = 80

RB = 2000
EB = 1280


def _node_kernel(x_ref, pos_ref, W_in_ref, b_in_ref, W_lin_ref, W_src_ref,
                 W_dst_ref, bp1_ref, Wp2_ref, bp2_ref, Wa1_ref, ba1_ref,
                 Wa2_ref, bb2_ref,
                 tsrc_ref, tdst_ref, s0_ref, v0_ref, wq_ref, cq_ref):
    x = x_ref[...]
    pos16 = pos_ref[...]
    h = jax.nn.relu(jnp.dot(x, W_in_ref[...], preferred_element_type=jnp.float32)
                    + b_in_ref[...])
    a_src = jnp.dot(h, W_src_ref[...], preferred_element_type=jnp.float32)
    a_dst = jnp.dot(h, W_dst_ref[...], preferred_element_type=jnp.float32)
    xl = jnp.dot(h, W_lin_ref[...], preferred_element_type=jnp.float32)
    Wa1 = Wa1_ref[...]
    bsrc = jnp.dot(a_src, Wa1, preferred_element_type=jnp.float32)
    bdst = jnp.dot(a_dst, Wa1, preferred_element_type=jnp.float32)
    zpad = jnp.zeros((pos16.shape[0], DS - PW - AH - D), dtype=jnp.float32)
    tsrc_ref[...] = jnp.concatenate([pos16, bsrc, xl, zpad], axis=1)
    tdst_ref[...] = jnp.concatenate([pos16, bdst, zpad], axis=1)
    Wp2 = Wp2_ref[...]
    wq_ref[...] = jnp.dot(Wp2, Wa1, preferred_element_type=jnp.float32)
    d0 = jnp.dot(jax.nn.relu(bp1_ref[...]), Wp2,
                 preferred_element_type=jnp.float32) + bp2_ref[...]
    cq_ref[...] = jnp.concatenate(
        [jnp.dot(bp2_ref[...], Wa1, preferred_element_type=jnp.float32)
         + ba1_ref[...], d0], axis=1)
    u0 = bdst - bsrc + jnp.dot(d0, Wa1, preferred_element_type=jnp.float32) \
        + ba1_ref[...]
    alpha0 = jnp.dot(jax.nn.relu(u0), Wa2_ref[...],
                     preferred_element_type=jnp.float32) + bb2_ref[...]
    s0 = jnp.exp(alpha0)
    s0_ref[...] = s0
    v0_ref[...] = s0 * (xl + d0)


def _sc_gather_body(src_hbm, dst_hbm, tsrc_hbm, tdst_hbm,
                    gsrc_out, gdst_out,
                    sidx, didx, srows, drows, sem_a, sem_b):
    wid = lax.axis_index("s") * NC + lax.axis_index("c")
    epw = E // NW
    base0 = wid * epw

    def body(i, carry):
        base = base0 + i * CG
        pltpu.sync_copy(src_hbm.at[pl.ds(base, CG)], sidx)
        pltpu.sync_copy(dst_hbm.at[pl.ds(base, CG)], didx)
        cp_a = pltpu.async_copy(tsrc_hbm.at[sidx], srows, sem_a)
        cp_b = pltpu.async_copy(tdst_hbm.at[didx], drows, sem_b)
        cp_a.wait()
        cp_b.wait()
        pltpu.sync_copy(srows, gsrc_out.at[pl.ds(base, CG)])
        pltpu.sync_copy(drows, gdst_out.at[pl.ds(base, CG)])
        return carry

    lax.fori_loop(0, epw // CG, body, 0)


def _edge_kernel(gsrc_ref, gdst_ref, bp1_ref, Wp1_ref, Wp2_ref, bp2_ref,
                 wq_ref, cq_ref, Wa2_ref, bb2_ref, sv_ref):
    gsrc = gsrc_ref[...]
    gdst = gdst_ref[...]
    pdiff = gdst[:, :PW] - gsrc[:, :PW]
    bdiff = gdst[:, PW:PW + AH] - gsrc[:, PW:PW + AH]
    xls = gsrc[:, PW + AH:PW + AH + D]
    cq = cq_ref[...]
    t = jax.nn.relu(jnp.dot(pdiff, Wp1_ref[...],
                            preferred_element_type=jnp.float32) + bp1_ref[...])
    delta = jnp.dot(t, Wp2_ref[...], preferred_element_type=jnp.float32) \
        + bp2_ref[...]
    u = bdiff + jnp.dot(t, wq_ref[...], preferred_element_type=jnp.float32) \
        + cq[:, :AH]
    alpha = jnp.dot(jax.nn.relu(u), Wa2_ref[...],
                    preferred_element_type=jnp.float32) + bb2_ref[...]
    s = jnp.exp(alpha)
    sv_ref[0] = s
    sv_ref[1] = s * (xls + delta)


def _sc_scatter_body(dsti_hbm, sv_hbm, zeros_hbm, out_hbm,
                     idx_v, rows_v, table_sh):
    c = lax.axis_index("c")
    s = lax.axis_index("s")

    @pl.when(s == 0)
    def _():
        pltpu.sync_copy(zeros_hbm, table_sh)

    plsc.subcore_barrier()

    ept = E // NS
    base0 = s * ept

    def body(i, carry):
        base = base0 + i * CS
        pltpu.sync_copy(dsti_hbm.at[pl.ds(base, CS)], idx_v)
        pltpu.sync_copy(sv_hbm.at[c, pl.ds(base, CS)], rows_v)
        pltpu.sync_copy(rows_v, table_sh.at[idx_v], add=True)
        return carry

    lax.fori_loop(0, ept // CS, body, 0)
    plsc.subcore_barrier()

    rpt = 624
    pltpu.sync_copy(table_sh.at[pl.ds(s * rpt, rpt)],
                    out_hbm.at[c, pl.ds(s * rpt, rpt)])

    @pl.when(s == 0)
    def _():
        pltpu.sync_copy(table_sh.at[pl.ds(NS * rpt, N - NS * rpt)],
                        out_hbm.at[c, pl.ds(NS * rpt, N - NS * rpt)])


def _final_kernel(pd_ref, pn_ref, s0_ref, v0_ref, W_out_ref, b_out_ref,
                  out_ref):
    denom = pd_ref[0] + s0_ref[...]
    num = pn_ref[0] + v0_ref[...]
    y = num / (denom + 1e-16)
    out_ref[...] = jax.nn.relu(
        jnp.dot(y, W_out_ref[...], preferred_element_type=jnp.float32)
        + b_out_ref[...])


def kernel(x, pos, edge_index, W_in, b_in, W_lin, W_src, W_dst, Wp1, bp1,
           Wp2, bp2, Wa1, ba1, Wa2, bb2, W_out, b_out):
    f32 = jnp.float32
    pos16 = jnp.pad(pos, ((0, 0), (0, PW - 3)))
    Wp1_16 = jnp.pad(Wp1, ((0, PW - 3), (0, 0)))
    src = edge_index[0].astype(jnp.int32)
    dst = edge_index[1].astype(jnp.int32)
    b_in2 = b_in.reshape(1, D)
    bp1_2 = bp1.reshape(1, AH)
    bp2_2 = bp2.reshape(1, D)
    ba1_2 = ba1.reshape(1, AH)
    bb2_2 = bb2.reshape(1, D)
    b_out2 = b_out.reshape(1, D)

    nsteps = N // RB
    full = lambda shp: pl.BlockSpec(shp, lambda i: tuple(0 for _ in shp))
    rows = lambda w: pl.BlockSpec((RB, w), lambda i: (i, 0))

    tsrc, tdst, s0, v0, wq, cq = pl.pallas_call(
        _node_kernel,
        grid=(nsteps,),
        in_specs=[rows(D), rows(PW), full((D, D)), full((1, D)),
                  full((D, D)), full((D, D)), full((D, D)), full((1, AH)),
                  full((AH, D)), full((1, D)), full((D, AH)), full((1, AH)),
                  full((AH, D)), full((1, D))],
        out_specs=[rows(DS), rows(DD), rows(D), rows(D),
                   full((AH, AH)), full((1, AH + D))],
        out_shape=[jax.ShapeDtypeStruct((N, DS), f32),
                   jax.ShapeDtypeStruct((N, DD), f32),
                   jax.ShapeDtypeStruct((N, D), f32),
                   jax.ShapeDtypeStruct((N, D), f32),
                   jax.ShapeDtypeStruct((AH, AH), f32),
                   jax.ShapeDtypeStruct((1, AH + D), f32)],
    )(x, pos16, W_in, b_in2, W_lin, W_src, W_dst, bp1_2, Wp2, bp2_2,
      Wa1, ba1_2, Wa2, bb2_2)

    mesh = plsc.VectorSubcoreMesh(core_axis_name="c", subcore_axis_name="s")

    gsrc, gdst = pl.kernel(
        _sc_gather_body,
        out_type=[jax.ShapeDtypeStruct((E, DS), f32),
                  jax.ShapeDtypeStruct((E, DD), f32)],
        mesh=mesh,
        scratch_types=[pltpu.VMEM((CG,), jnp.int32),
                       pltpu.VMEM((CG,), jnp.int32),
                       pltpu.VMEM((CG, DS), f32),
                       pltpu.VMEM((CG, DD), f32),
                       pltpu.SemaphoreType.DMA,
                       pltpu.SemaphoreType.DMA],
    )(src, dst, tsrc, tdst)

    esteps = E // EB
    erows = lambda w: pl.BlockSpec((EB, w), lambda i: (i, 0))
    sv = pl.pallas_call(
        _edge_kernel,
        grid=(esteps,),
        in_specs=[erows(DS), erows(DD), full((1, AH)), full((PW, AH)),
                  full((AH, D)), full((1, D)), full((AH, AH)),
                  full((1, AH + D)), full((AH, D)), full((1, D))],
        out_specs=[pl.BlockSpec((2, EB, D), lambda i: (0, i, 0))],
        out_shape=[jax.ShapeDtypeStruct((2, E, D), f32)],
    )(gsrc, gdst, bp1_2, Wp1_16, Wp2, bp2_2, wq, cq, Wa2, bb2_2)[0]

    zeros = jnp.zeros((N, D), f32)
    parts = pl.kernel(
        _sc_scatter_body,
        out_type=jax.ShapeDtypeStruct((2, N, D), f32),
        mesh=mesh,
        scratch_types=[pltpu.VMEM((CS,), jnp.int32),
                       pltpu.VMEM((CS, D), f32),
                       pltpu.VMEM_SHARED((N, D), f32)],
    )(dst, sv, zeros)

    out = pl.pallas_call(
        _final_kernel,
        grid=(nsteps,),
        in_specs=[pl.BlockSpec((1, RB, D), lambda i: (0, i, 0)),
                  pl.BlockSpec((1, RB, D), lambda i: (1, i, 0)),
                  rows(D), rows(D), full((D, D)), full((1, D))],
        out_specs=[rows(D)],
        out_shape=[jax.ShapeDtypeStruct((N, D), f32)],
    )(parts, parts, s0, v0, W_out, b_out2)[0]
    return out


# speedup vs baseline: 8.4765x; 1.3096x over previous
import functools

import jax
import jax.numpy as jnp
from jax import lax
from jax.experimental import pallas as pl
from jax.experimental.pallas import tpu as pltpu
from jax.experimental.pallas import tpu_sc as plsc

N = 10000
E = 320000
D = 128
AH = 64
PW = 16
DS = 256
DD = 128

NC = 2
NS = 16
NW = NC * NS

CG = 40
CS = 80

RB = 2000
EB = 1280


def _node_kernel(x_ref, pos_ref, W_in_ref, b_in_ref, W_lin_ref, W_src_ref,
                 W_dst_ref, bp1_ref, Wp2_ref, bp2_ref, Wa1_ref, ba1_ref,
                 Wa2_ref, bb2_ref,
                 tsrc_ref, tdst_ref, s0_ref, v0_ref, wq_ref, cq_ref):
    x = x_ref[...]
    pos16 = pos_ref[...]
    h = jax.nn.relu(jnp.dot(x, W_in_ref[...], preferred_element_type=jnp.float32)
                    + b_in_ref[...])
    a_src = jnp.dot(h, W_src_ref[...], preferred_element_type=jnp.float32)
    a_dst = jnp.dot(h, W_dst_ref[...], preferred_element_type=jnp.float32)
    xl = jnp.dot(h, W_lin_ref[...], preferred_element_type=jnp.float32)
    Wa1 = Wa1_ref[...]
    bsrc = jnp.dot(a_src, Wa1, preferred_element_type=jnp.float32)
    bdst = jnp.dot(a_dst, Wa1, preferred_element_type=jnp.float32)
    zpad = jnp.zeros((pos16.shape[0], DS - PW - AH - D), dtype=jnp.float32)
    tsrc_ref[...] = jnp.concatenate([pos16, bsrc, xl, zpad], axis=1)
    tdst_ref[...] = jnp.concatenate([pos16, bdst, zpad], axis=1)
    Wp2 = Wp2_ref[...]
    wq_ref[...] = jnp.dot(Wp2, Wa1, preferred_element_type=jnp.float32)
    d0 = jnp.dot(jax.nn.relu(bp1_ref[...]), Wp2,
                 preferred_element_type=jnp.float32) + bp2_ref[...]
    cq_ref[...] = jnp.concatenate(
        [jnp.dot(bp2_ref[...], Wa1, preferred_element_type=jnp.float32)
         + ba1_ref[...], d0], axis=1)
    u0 = bdst - bsrc + jnp.dot(d0, Wa1, preferred_element_type=jnp.float32) \
        + ba1_ref[...]
    alpha0 = jnp.dot(jax.nn.relu(u0), Wa2_ref[...],
                     preferred_element_type=jnp.float32) + bb2_ref[...]
    s0 = jnp.exp(alpha0)
    s0_ref[...] = s0
    v0_ref[...] = s0 * (xl + d0)


def _sc_gather_body(src_hbm, dst_hbm, tsrc_hbm, tdst_hbm,
                    gsrc_out, gdst_out,
                    sidx, didx, sr0, sr1, dr0, dr1,
                    gsem0, gsem1, wsem0, wsem1):
    wid = lax.axis_index("s") * NC + lax.axis_index("c")
    epw = E // NW
    nch = epw // CG
    base0 = wid * epw
    pltpu.sync_copy(src_hbm.at[pl.ds(base0, epw)], sidx)
    pltpu.sync_copy(dst_hbm.at[pl.ds(base0, epw)], didx)

    def g_start(c, srows, drows, sem):
        off = pl.multiple_of(c * CG, 8)
        pltpu.async_copy(tsrc_hbm.at[sidx.at[pl.ds(off, CG)]], srows, sem)
        pltpu.async_copy(tdst_hbm.at[didx.at[pl.ds(off, CG)]], drows, sem)

    def g_wait(srows, drows, sem):
        pltpu.make_async_copy(tsrc_hbm.at[pl.ds(0, CG)], srows, sem).wait()
        pltpu.make_async_copy(tdst_hbm.at[pl.ds(0, CG)], drows, sem).wait()

    def w_start(c, srows, drows, sem):
        base = base0 + pl.multiple_of(c * CG, 8)
        pltpu.async_copy(srows, gsrc_out.at[pl.ds(base, CG)], sem)
        pltpu.async_copy(drows, gdst_out.at[pl.ds(base, CG)], sem)

    def w_wait(srows, drows, sem):
        pltpu.make_async_copy(srows, gsrc_out.at[pl.ds(0, CG)], sem).wait()
        pltpu.make_async_copy(drows, gdst_out.at[pl.ds(0, CG)], sem).wait()

    g_start(0, sr0, dr0, gsem0)
    g_start(1, sr1, dr1, gsem1)
    g_wait(sr0, dr0, gsem0)
    w_start(0, sr0, dr0, wsem0)
    g_wait(sr1, dr1, gsem1)
    w_start(1, sr1, dr1, wsem1)

    def body(k, carry):
        c = 2 * k + 2
        w_wait(sr0, dr0, wsem0)
        g_start(c, sr0, dr0, gsem0)
        w_wait(sr1, dr1, wsem1)
        g_start(c + 1, sr1, dr1, gsem1)
        g_wait(sr0, dr0, gsem0)
        w_start(c, sr0, dr0, wsem0)
        g_wait(sr1, dr1, gsem1)
        w_start(c + 1, sr1, dr1, wsem1)
        return carry

    lax.fori_loop(0, nch // 2 - 1, body, 0)
    w_wait(sr0, dr0, wsem0)
    w_wait(sr1, dr1, wsem1)


def _edge_kernel(gsrc_ref, gdst_ref, bp1_ref, Wp1_ref, Wp2_ref, bp2_ref,
                 wq_ref, cq_ref, Wa2_ref, bb2_ref, sv_ref):
    gsrc = gsrc_ref[...]
    gdst = gdst_ref[...]
    pdiff = gdst[:, :PW] - gsrc[:, :PW]
    bdiff = gdst[:, PW:PW + AH] - gsrc[:, PW:PW + AH]
    xls = gsrc[:, PW + AH:PW + AH + D]
    cq = cq_ref[...]
    t = jax.nn.relu(jnp.dot(pdiff, Wp1_ref[...],
                            preferred_element_type=jnp.float32) + bp1_ref[...])
    delta = jnp.dot(t, Wp2_ref[...], preferred_element_type=jnp.float32) \
        + bp2_ref[...]
    u = bdiff + jnp.dot(t, wq_ref[...], preferred_element_type=jnp.float32) \
        + cq[:, :AH]
    alpha = jnp.dot(jax.nn.relu(u), Wa2_ref[...],
                    preferred_element_type=jnp.float32) + bb2_ref[...]
    s = jnp.exp(alpha)
    sv_ref[0] = s
    sv_ref[1] = s * (xls + delta)


def _sc_scatter_body(dsti_hbm, sv_hbm, zeros_hbm, out_hbm,
                     idx0, idx1, rows0, rows1, lsem0, lsem1, ssem0, ssem1,
                     table_sh):
    c = lax.axis_index("c")
    s = lax.axis_index("s")

    @pl.when(s == 0)
    def _():
        pltpu.sync_copy(zeros_hbm, table_sh)

    ept = E // NS
    nch = ept // CS
    base0 = s * ept
    plsc.subcore_barrier()

    def l_start(ch, idx, rows, sem):
        base = base0 + pl.multiple_of(ch * CS, 8)
        pltpu.async_copy(dsti_hbm.at[pl.ds(base, CS)], idx, sem)
        pltpu.async_copy(sv_hbm.at[c, pl.ds(base, CS)], rows, sem)

    def l_wait(idx, rows, sem):
        pltpu.make_async_copy(dsti_hbm.at[pl.ds(0, CS)], idx, sem).wait()
        pltpu.make_async_copy(sv_hbm.at[0, pl.ds(0, CS)], rows, sem).wait()

    def s_start(idx, rows, sem):
        pltpu.async_copy(rows, table_sh.at[idx], sem, add=True)

    def s_wait(rows, sem):
        pltpu.make_async_copy(rows, table_sh.at[pl.ds(0, CS)], sem).wait()

    l_start(0, idx0, rows0, lsem0)
    l_start(1, idx1, rows1, lsem1)
    l_wait(idx0, rows0, lsem0)
    s_start(idx0, rows0, ssem0)
    l_wait(idx1, rows1, lsem1)
    s_start(idx1, rows1, ssem1)

    def body(k, carry):
        ch = 2 * k + 2
        s_wait(rows0, ssem0)
        l_start(ch, idx0, rows0, lsem0)
        s_wait(rows1, ssem1)
        l_start(ch + 1, idx1, rows1, lsem1)
        l_wait(idx0, rows0, lsem0)
        s_start(idx0, rows0, ssem0)
        l_wait(idx1, rows1, lsem1)
        s_start(idx1, rows1, ssem1)
        return carry

    lax.fori_loop(0, nch // 2 - 1, body, 0)
    s_wait(rows0, ssem0)
    s_wait(rows1, ssem1)
    plsc.subcore_barrier()

    rpt = 624
    pltpu.sync_copy(table_sh.at[pl.ds(s * rpt, rpt)],
                    out_hbm.at[c, pl.ds(s * rpt, rpt)])

    @pl.when(s == 0)
    def _():
        pltpu.sync_copy(table_sh.at[pl.ds(NS * rpt, N - NS * rpt)],
                        out_hbm.at[c, pl.ds(NS * rpt, N - NS * rpt)])


def _final_kernel(pd_ref, pn_ref, s0_ref, v0_ref, W_out_ref, b_out_ref,
                  out_ref):
    denom = pd_ref[0] + s0_ref[...]
    num = pn_ref[0] + v0_ref[...]
    y = num / (denom + 1e-16)
    out_ref[...] = jax.nn.relu(
        jnp.dot(y, W_out_ref[...], preferred_element_type=jnp.float32)
        + b_out_ref[...])


def kernel(x, pos, edge_index, W_in, b_in, W_lin, W_src, W_dst, Wp1, bp1,
           Wp2, bp2, Wa1, ba1, Wa2, bb2, W_out, b_out):
    f32 = jnp.float32
    pos16 = jnp.pad(pos, ((0, 0), (0, PW - 3)))
    Wp1_16 = jnp.pad(Wp1, ((0, PW - 3), (0, 0)))
    src = edge_index[0].astype(jnp.int32)
    dst = edge_index[1].astype(jnp.int32)
    b_in2 = b_in.reshape(1, D)
    bp1_2 = bp1.reshape(1, AH)
    bp2_2 = bp2.reshape(1, D)
    ba1_2 = ba1.reshape(1, AH)
    bb2_2 = bb2.reshape(1, D)
    b_out2 = b_out.reshape(1, D)

    nsteps = N // RB
    full = lambda shp: pl.BlockSpec(shp, lambda i: tuple(0 for _ in shp))
    rows = lambda w: pl.BlockSpec((RB, w), lambda i: (i, 0))

    tsrc, tdst, s0, v0, wq, cq = pl.pallas_call(
        _node_kernel,
        grid=(nsteps,),
        in_specs=[rows(D), rows(PW), full((D, D)), full((1, D)),
                  full((D, D)), full((D, D)), full((D, D)), full((1, AH)),
                  full((AH, D)), full((1, D)), full((D, AH)), full((1, AH)),
                  full((AH, D)), full((1, D))],
        out_specs=[rows(DS), rows(DD), rows(D), rows(D),
                   full((AH, AH)), full((1, AH + D))],
        out_shape=[jax.ShapeDtypeStruct((N, DS), f32),
                   jax.ShapeDtypeStruct((N, DD), f32),
                   jax.ShapeDtypeStruct((N, D), f32),
                   jax.ShapeDtypeStruct((N, D), f32),
                   jax.ShapeDtypeStruct((AH, AH), f32),
                   jax.ShapeDtypeStruct((1, AH + D), f32)],
    )(x, pos16, W_in, b_in2, W_lin, W_src, W_dst, bp1_2, Wp2, bp2_2,
      Wa1, ba1_2, Wa2, bb2_2)

    mesh = plsc.VectorSubcoreMesh(core_axis_name="c", subcore_axis_name="s")

    gsrc, gdst = pl.kernel(
        _sc_gather_body,
        out_type=[jax.ShapeDtypeStruct((E, DS), f32),
                  jax.ShapeDtypeStruct((E, DD), f32)],
        mesh=mesh,
        scratch_types=[pltpu.VMEM((E // NW,), jnp.int32),
                       pltpu.VMEM((E // NW,), jnp.int32),
                       pltpu.VMEM((CG, DS), f32),
                       pltpu.VMEM((CG, DS), f32),
                       pltpu.VMEM((CG, DD), f32),
                       pltpu.VMEM((CG, DD), f32),
                       pltpu.SemaphoreType.DMA,
                       pltpu.SemaphoreType.DMA,
                       pltpu.SemaphoreType.DMA,
                       pltpu.SemaphoreType.DMA],
    )(src, dst, tsrc, tdst)

    esteps = E // EB
    erows = lambda w: pl.BlockSpec((EB, w), lambda i: (i, 0))
    sv = pl.pallas_call(
        _edge_kernel,
        grid=(esteps,),
        in_specs=[erows(DS), erows(DD), full((1, AH)), full((PW, AH)),
                  full((AH, D)), full((1, D)), full((AH, AH)),
                  full((1, AH + D)), full((AH, D)), full((1, D))],
        out_specs=[pl.BlockSpec((2, EB, D), lambda i: (0, i, 0))],
        out_shape=[jax.ShapeDtypeStruct((2, E, D), f32)],
    )(gsrc, gdst, bp1_2, Wp1_16, Wp2, bp2_2, wq, cq, Wa2, bb2_2)[0]

    zeros = jnp.zeros((N, D), f32)
    parts = pl.kernel(
        _sc_scatter_body,
        out_type=jax.ShapeDtypeStruct((2, N, D), f32),
        mesh=mesh,
        scratch_types=[pltpu.VMEM((CS,), jnp.int32),
                       pltpu.VMEM((CS,), jnp.int32),
                       pltpu.VMEM((CS, D), f32),
                       pltpu.VMEM((CS, D), f32),
                       pltpu.SemaphoreType.DMA,
                       pltpu.SemaphoreType.DMA,
                       pltpu.SemaphoreType.DMA,
                       pltpu.SemaphoreType.DMA,
                       pltpu.VMEM_SHARED((N, D), f32)],
    )(dst, sv, zeros)

    out = pl.pallas_call(
        _final_kernel,
        grid=(nsteps,),
        in_specs=[pl.BlockSpec((1, RB, D), lambda i: (0, i, 0)),
                  pl.BlockSpec((1, RB, D), lambda i: (1, i, 0)),
                  rows(D), rows(D), full((D, D)), full((1, D))],
        out_specs=[rows(D)],
        out_shape=[jax.ShapeDtypeStruct((N, D), f32)],
    )(parts, parts, s0, v0, W_out, b_out2)[0]
    return out


# speedup vs baseline: 8.8916x; 1.0490x over previous
import functools

import jax
import jax.numpy as jnp
from jax import lax
from jax.experimental import pallas as pl
from jax.experimental.pallas import tpu as pltpu
from jax.experimental.pallas import tpu_sc as plsc

N = 10000
E = 320000
D = 128
AH = 64
PW = 16
DS = 256
DD = 128

NC = 2
NS = 16
NW = NC * NS

CG = 80
CS = 80
CL = 2 * CS

RB = 2000
EB = 1280


def _node_kernel(x_ref, pos_ref, W_in_ref, b_in_ref, W_lin_ref, W_src_ref,
                 W_dst_ref, bp1_ref, Wp2_ref, bp2_ref, Wa1_ref, ba1_ref,
                 Wa2_ref, bb2_ref,
                 tsrc_ref, tdst_ref, s0_ref, v0_ref, wq_ref, cq_ref):
    x = x_ref[...]
    pos16 = pos_ref[...]
    h = jax.nn.relu(jnp.dot(x, W_in_ref[...], preferred_element_type=jnp.float32)
                    + b_in_ref[...])
    a_src = jnp.dot(h, W_src_ref[...], preferred_element_type=jnp.float32)
    a_dst = jnp.dot(h, W_dst_ref[...], preferred_element_type=jnp.float32)
    xl = jnp.dot(h, W_lin_ref[...], preferred_element_type=jnp.float32)
    Wa1 = Wa1_ref[...]
    bsrc = jnp.dot(a_src, Wa1, preferred_element_type=jnp.float32)
    bdst = jnp.dot(a_dst, Wa1, preferred_element_type=jnp.float32)
    zpad = jnp.zeros((pos16.shape[0], DS - PW - AH - D), dtype=jnp.float32)
    tsrc_ref[...] = jnp.concatenate([pos16, bsrc, xl, zpad], axis=1)
    tdst_ref[...] = jnp.concatenate([pos16, bdst, zpad], axis=1)
    Wp2 = Wp2_ref[...]
    wq_ref[...] = jnp.dot(Wp2, Wa1, preferred_element_type=jnp.float32)
    d0 = jnp.dot(jax.nn.relu(bp1_ref[...]), Wp2,
                 preferred_element_type=jnp.float32) + bp2_ref[...]
    cq_ref[...] = jnp.concatenate(
        [jnp.dot(bp2_ref[...], Wa1, preferred_element_type=jnp.float32)
         + ba1_ref[...], d0], axis=1)
    u0 = bdst - bsrc + jnp.dot(d0, Wa1, preferred_element_type=jnp.float32) \
        + ba1_ref[...]
    alpha0 = jnp.dot(jax.nn.relu(u0), Wa2_ref[...],
                     preferred_element_type=jnp.float32) + bb2_ref[...]
    s0 = jnp.exp(alpha0)
    s0_ref[...] = s0
    v0_ref[...] = s0 * (xl + d0)


def _sc_gather_body(src_hbm, dst_hbm, tsrc_hbm, tdst_hbm,
                    gsrc_out, gdst_out,
                    sidx, didx, sr0, sr1, dr0, dr1,
                    gsem0, gsem1, wsem0, wsem1):
    wid = lax.axis_index("s") * NC + lax.axis_index("c")
    epw = E // NW
    nch = epw // CG
    base0 = wid * epw
    pltpu.sync_copy(src_hbm.at[pl.ds(base0, epw)], sidx)
    pltpu.sync_copy(dst_hbm.at[pl.ds(base0, epw)], didx)

    def g_start(c, srows, drows, sem):
        off = pl.multiple_of(c * CG, 8)
        pltpu.async_copy(tsrc_hbm.at[sidx.at[pl.ds(off, CG)]], srows, sem)
        pltpu.async_copy(tdst_hbm.at[didx.at[pl.ds(off, CG)]], drows, sem)

    def g_wait(srows, drows, sem):
        pltpu.make_async_copy(tsrc_hbm.at[pl.ds(0, CG)], srows, sem).wait()
        pltpu.make_async_copy(tdst_hbm.at[pl.ds(0, CG)], drows, sem).wait()

    def w_start(c, srows, drows, sem):
        base = base0 + pl.multiple_of(c * CG, 8)
        pltpu.async_copy(srows, gsrc_out.at[pl.ds(base, CG)], sem)
        pltpu.async_copy(drows, gdst_out.at[pl.ds(base, CG)], sem)

    def w_wait(srows, drows, sem):
        pltpu.make_async_copy(srows, gsrc_out.at[pl.ds(0, CG)], sem).wait()
        pltpu.make_async_copy(drows, gdst_out.at[pl.ds(0, CG)], sem).wait()

    g_start(0, sr0, dr0, gsem0)
    g_start(1, sr1, dr1, gsem1)
    g_wait(sr0, dr0, gsem0)
    w_start(0, sr0, dr0, wsem0)
    g_wait(sr1, dr1, gsem1)
    w_start(1, sr1, dr1, wsem1)

    def body(k, carry):
        c = 2 * k + 2
        w_wait(sr0, dr0, wsem0)
        g_start(c, sr0, dr0, gsem0)
        w_wait(sr1, dr1, wsem1)
        g_start(c + 1, sr1, dr1, gsem1)
        g_wait(sr0, dr0, gsem0)
        w_start(c, sr0, dr0, wsem0)
        g_wait(sr1, dr1, gsem1)
        w_start(c + 1, sr1, dr1, wsem1)
        return carry

    lax.fori_loop(0, (nch - 2) // 2, body, 0)
    if nch % 2:
        w_wait(sr0, dr0, wsem0)
        g_start(nch - 1, sr0, dr0, gsem0)
        g_wait(sr0, dr0, gsem0)
        w_start(nch - 1, sr0, dr0, wsem0)
    w_wait(sr0, dr0, wsem0)
    w_wait(sr1, dr1, wsem1)


def _edge_kernel(gsrc_ref, gdst_ref, bp1_ref, Wp1_ref, Wp2_ref, bp2_ref,
                 wq_ref, cq_ref, Wa2_ref, bb2_ref, sv_ref):
    gsrc = gsrc_ref[...]
    gdst = gdst_ref[...]
    pdiff = gdst[:, :PW] - gsrc[:, :PW]
    bdiff = gdst[:, PW:PW + AH] - gsrc[:, PW:PW + AH]
    xls = gsrc[:, PW + AH:PW + AH + D]
    cq = cq_ref[...]
    t = jax.nn.relu(jnp.dot(pdiff, Wp1_ref[...],
                            preferred_element_type=jnp.float32) + bp1_ref[...])
    delta = jnp.dot(t, Wp2_ref[...], preferred_element_type=jnp.float32) \
        + bp2_ref[...]
    u = bdiff + jnp.dot(t, wq_ref[...], preferred_element_type=jnp.float32) \
        + cq[:, :AH]
    alpha = jnp.dot(jax.nn.relu(u), Wa2_ref[...],
                    preferred_element_type=jnp.float32) + bb2_ref[...]
    s = jnp.exp(alpha)
    sv_ref[0] = s
    sv_ref[1] = s * (xls + delta)


def _sc_scatter_body(dsti_hbm, sv_hbm, zeros_hbm, out_hbm,
                     idx0a, idx0b, idx1a, idx1b, rows0, rows1,
                     lsem0, lsem1, ssem0, ssem1, table_sh):
    c = lax.axis_index("c")
    s = lax.axis_index("s")

    @pl.when(s == 0)
    def _():
        pltpu.sync_copy(zeros_hbm, table_sh)

    ept = E // NS
    nch = ept // CL
    base0 = s * ept
    plsc.subcore_barrier()

    def l_start(ch, idxa, idxb, rows, sem):
        base = base0 + pl.multiple_of(ch * CL, 8)
        pltpu.async_copy(dsti_hbm.at[pl.ds(base, CS)], idxa, sem)
        pltpu.async_copy(dsti_hbm.at[pl.ds(base + CS, CS)], idxb, sem)
        pltpu.async_copy(sv_hbm.at[c, pl.ds(base, CL)], rows, sem)

    def l_wait(idxa, idxb, rows, sem):
        pltpu.make_async_copy(dsti_hbm.at[pl.ds(0, CS)], idxa, sem).wait()
        pltpu.make_async_copy(dsti_hbm.at[pl.ds(0, CS)], idxb, sem).wait()
        pltpu.make_async_copy(sv_hbm.at[0, pl.ds(0, CL)], rows, sem).wait()

    def s_start(idxa, idxb, rows, sem):
        pltpu.async_copy(rows.at[pl.ds(0, CS)], table_sh.at[idxa], sem,
                         add=True)
        pltpu.async_copy(rows.at[pl.ds(CS, CS)], table_sh.at[idxb], sem,
                         add=True)

    def s_wait(rows, sem):
        pltpu.make_async_copy(rows, table_sh.at[pl.ds(0, CL)], sem).wait()

    l_start(0, idx0a, idx0b, rows0, lsem0)
    l_start(1, idx1a, idx1b, rows1, lsem1)
    l_wait(idx0a, idx0b, rows0, lsem0)
    s_start(idx0a, idx0b, rows0, ssem0)
    l_wait(idx1a, idx1b, rows1, lsem1)
    s_start(idx1a, idx1b, rows1, ssem1)

    def body(k, carry):
        ch = 2 * k + 2
        s_wait(rows0, ssem0)
        l_start(ch, idx0a, idx0b, rows0, lsem0)
        s_wait(rows1, ssem1)
        l_start(ch + 1, idx1a, idx1b, rows1, lsem1)
        l_wait(idx0a, idx0b, rows0, lsem0)
        s_start(idx0a, idx0b, rows0, ssem0)
        l_wait(idx1a, idx1b, rows1, lsem1)
        s_start(idx1a, idx1b, rows1, ssem1)
        return carry

    lax.fori_loop(0, (nch - 2) // 2, body, 0)
    if nch % 2:
        s_wait(rows0, ssem0)
        l_start(nch - 1, idx0a, idx0b, rows0, lsem0)
        l_wait(idx0a, idx0b, rows0, lsem0)
        s_start(idx0a, idx0b, rows0, ssem0)
    s_wait(rows0, ssem0)
    s_wait(rows1, ssem1)
    plsc.subcore_barrier()

    rpt = 624
    pltpu.sync_copy(table_sh.at[pl.ds(s * rpt, rpt)],
                    out_hbm.at[c, pl.ds(s * rpt, rpt)])

    @pl.when(s == 0)
    def _():
        pltpu.sync_copy(table_sh.at[pl.ds(NS * rpt, N - NS * rpt)],
                        out_hbm.at[c, pl.ds(NS * rpt, N - NS * rpt)])


def _final_kernel(pd_ref, pn_ref, s0_ref, v0_ref, W_out_ref, b_out_ref,
                  out_ref):
    denom = pd_ref[0] + s0_ref[...]
    num = pn_ref[0] + v0_ref[...]
    y = num / (denom + 1e-16)
    out_ref[...] = jax.nn.relu(
        jnp.dot(y, W_out_ref[...], preferred_element_type=jnp.float32)
        + b_out_ref[...])


def kernel(x, pos, edge_index, W_in, b_in, W_lin, W_src, W_dst, Wp1, bp1,
           Wp2, bp2, Wa1, ba1, Wa2, bb2, W_out, b_out):
    f32 = jnp.float32
    pos16 = jnp.pad(pos, ((0, 0), (0, PW - 3)))
    Wp1_16 = jnp.pad(Wp1, ((0, PW - 3), (0, 0)))
    src = edge_index[0].astype(jnp.int32)
    dst = edge_index[1].astype(jnp.int32)
    b_in2 = b_in.reshape(1, D)
    bp1_2 = bp1.reshape(1, AH)
    bp2_2 = bp2.reshape(1, D)
    ba1_2 = ba1.reshape(1, AH)
    bb2_2 = bb2.reshape(1, D)
    b_out2 = b_out.reshape(1, D)

    nsteps = N // RB
    full = lambda shp: pl.BlockSpec(shp, lambda i: tuple(0 for _ in shp))
    rows = lambda w: pl.BlockSpec((RB, w), lambda i: (i, 0))

    tsrc, tdst, s0, v0, wq, cq = pl.pallas_call(
        _node_kernel,
        grid=(nsteps,),
        in_specs=[rows(D), rows(PW), full((D, D)), full((1, D)),
                  full((D, D)), full((D, D)), full((D, D)), full((1, AH)),
                  full((AH, D)), full((1, D)), full((D, AH)), full((1, AH)),
                  full((AH, D)), full((1, D))],
        out_specs=[rows(DS), rows(DD), rows(D), rows(D),
                   full((AH, AH)), full((1, AH + D))],
        out_shape=[jax.ShapeDtypeStruct((N, DS), f32),
                   jax.ShapeDtypeStruct((N, DD), f32),
                   jax.ShapeDtypeStruct((N, D), f32),
                   jax.ShapeDtypeStruct((N, D), f32),
                   jax.ShapeDtypeStruct((AH, AH), f32),
                   jax.ShapeDtypeStruct((1, AH + D), f32)],
    )(x, pos16, W_in, b_in2, W_lin, W_src, W_dst, bp1_2, Wp2, bp2_2,
      Wa1, ba1_2, Wa2, bb2_2)

    mesh = plsc.VectorSubcoreMesh(core_axis_name="c", subcore_axis_name="s")

    gsrc, gdst = pl.kernel(
        _sc_gather_body,
        out_type=[jax.ShapeDtypeStruct((E, DS), f32),
                  jax.ShapeDtypeStruct((E, DD), f32)],
        mesh=mesh,
        scratch_types=[pltpu.VMEM((E // NW,), jnp.int32),
                       pltpu.VMEM((E // NW,), jnp.int32),
                       pltpu.VMEM((CG, DS), f32),
                       pltpu.VMEM((CG, DS), f32),
                       pltpu.VMEM((CG, DD), f32),
                       pltpu.VMEM((CG, DD), f32),
                       pltpu.SemaphoreType.DMA,
                       pltpu.SemaphoreType.DMA,
                       pltpu.SemaphoreType.DMA,
                       pltpu.SemaphoreType.DMA],
    )(src, dst, tsrc, tdst)

    esteps = E // EB
    erows = lambda w: pl.BlockSpec((EB, w), lambda i: (i, 0))
    sv = pl.pallas_call(
        _edge_kernel,
        grid=(esteps,),
        in_specs=[erows(DS), erows(DD), full((1, AH)), full((PW, AH)),
                  full((AH, D)), full((1, D)), full((AH, AH)),
                  full((1, AH + D)), full((AH, D)), full((1, D))],
        out_specs=[pl.BlockSpec((2, EB, D), lambda i: (0, i, 0))],
        out_shape=[jax.ShapeDtypeStruct((2, E, D), f32)],
    )(gsrc, gdst, bp1_2, Wp1_16, Wp2, bp2_2, wq, cq, Wa2, bb2_2)[0]

    zeros = jnp.zeros((N, D), f32)
    parts = pl.kernel(
        _sc_scatter_body,
        out_type=jax.ShapeDtypeStruct((2, N, D), f32),
        mesh=mesh,
        scratch_types=[pltpu.VMEM((CS,), jnp.int32),
                       pltpu.VMEM((CS,), jnp.int32),
                       pltpu.VMEM((CS,), jnp.int32),
                       pltpu.VMEM((CS,), jnp.int32),
                       pltpu.VMEM((CL, D), f32),
                       pltpu.VMEM((CL, D), f32),
                       pltpu.SemaphoreType.DMA,
                       pltpu.SemaphoreType.DMA,
                       pltpu.SemaphoreType.DMA,
                       pltpu.SemaphoreType.DMA,
                       pltpu.VMEM_SHARED((N, D), f32)],
    )(dst, sv, zeros)

    out = pl.pallas_call(
        _final_kernel,
        grid=(nsteps,),
        in_specs=[pl.BlockSpec((1, RB, D), lambda i: (0, i, 0)),
                  pl.BlockSpec((1, RB, D), lambda i: (1, i, 0)),
                  rows(D), rows(D), full((D, D)), full((1, D))],
        out_specs=[rows(D)],
        out_shape=[jax.ShapeDtypeStruct((N, D), f32)],
    )(parts, parts, s0, v0, W_out, b_out2)[0]
    return out


# speedup vs baseline: 10.2769x; 1.1558x over previous
import functools

import jax
import jax.numpy as jnp
from jax import lax
from jax.experimental import pallas as pl
from jax.experimental.pallas import tpu as pltpu
from jax.experimental.pallas import tpu_sc as plsc

N = 10000
E = 320000
D = 128
AH = 64
PW = 16
DS = 256
DD = 128

NC = 2
NS = 16
NW = NC * NS

NH = 2
CG = 40
CS = 40
CL = 2 * CS

RB = 2000
EB = 1280


def _node_kernel(x_ref, pos_ref, W_in_ref, b_in_ref, W_lin_ref, W_src_ref,
                 W_dst_ref, bp1_ref, Wp2_ref, bp2_ref, Wa1_ref, ba1_ref,
                 Wa2_ref, bb2_ref,
                 tsrc_ref, tdst_ref, s0_ref, v0_ref, wq_ref, cq_ref):
    x = x_ref[...]
    pos16 = pos_ref[...]
    h = jax.nn.relu(jnp.dot(x, W_in_ref[...], preferred_element_type=jnp.float32)
                    + b_in_ref[...])
    a_src = jnp.dot(h, W_src_ref[...], preferred_element_type=jnp.float32)
    a_dst = jnp.dot(h, W_dst_ref[...], preferred_element_type=jnp.float32)
    xl = jnp.dot(h, W_lin_ref[...], preferred_element_type=jnp.float32)
    Wa1 = Wa1_ref[...]
    bsrc = jnp.dot(a_src, Wa1, preferred_element_type=jnp.float32)
    bdst = jnp.dot(a_dst, Wa1, preferred_element_type=jnp.float32)
    zpad = jnp.zeros((pos16.shape[0], DS - PW - AH - D), dtype=jnp.float32)
    tsrc_ref[...] = jnp.concatenate([pos16, bsrc, xl, zpad], axis=1)
    tdst_ref[...] = jnp.concatenate([pos16, bdst, zpad], axis=1)
    Wp2 = Wp2_ref[...]
    wq_ref[...] = jnp.dot(Wp2, Wa1, preferred_element_type=jnp.float32)
    d0 = jnp.dot(jax.nn.relu(bp1_ref[...]), Wp2,
                 preferred_element_type=jnp.float32) + bp2_ref[...]
    cq_ref[...] = jnp.concatenate(
        [jnp.dot(bp2_ref[...], Wa1, preferred_element_type=jnp.float32)
         + ba1_ref[...], d0], axis=1)
    u0 = bdst - bsrc + jnp.dot(d0, Wa1, preferred_element_type=jnp.float32) \
        + ba1_ref[...]
    alpha0 = jnp.dot(jax.nn.relu(u0), Wa2_ref[...],
                     preferred_element_type=jnp.float32) + bb2_ref[...]
    s0 = jnp.exp(alpha0)
    s0_ref[...] = s0
    v0_ref[...] = s0 * (xl + d0)


def _sc_gather_body(src_hbm, dst_hbm, tsrc_hbm, tdst_hbm,
                    gsrc_out, gdst_out,
                    sidx, didx, sr0, sr1, dr0, dr1,
                    gsem0, gsem1, wsem0, wsem1):
    wid = lax.axis_index("s") * NC + lax.axis_index("c")
    epw = src_hbm.shape[0] // NW
    nch = epw // CG
    base0 = wid * epw
    pltpu.sync_copy(src_hbm.at[pl.ds(base0, epw)], sidx)
    pltpu.sync_copy(dst_hbm.at[pl.ds(base0, epw)], didx)

    def g_start(c, srows, drows, sem):
        off = pl.multiple_of(c * CG, 8)
        pltpu.async_copy(tsrc_hbm.at[sidx.at[pl.ds(off, CG)]], srows, sem)
        pltpu.async_copy(tdst_hbm.at[didx.at[pl.ds(off, CG)]], drows, sem)

    def g_wait(srows, drows, sem):
        pltpu.make_async_copy(tsrc_hbm.at[pl.ds(0, CG)], srows, sem).wait()
        pltpu.make_async_copy(tdst_hbm.at[pl.ds(0, CG)], drows, sem).wait()

    def w_start(c, srows, drows, sem):
        base = base0 + pl.multiple_of(c * CG, 8)
        pltpu.async_copy(srows, gsrc_out.at[pl.ds(base, CG)], sem)
        pltpu.async_copy(drows, gdst_out.at[pl.ds(base, CG)], sem)

    def w_wait(srows, drows, sem):
        pltpu.make_async_copy(srows, gsrc_out.at[pl.ds(0, CG)], sem).wait()
        pltpu.make_async_copy(drows, gdst_out.at[pl.ds(0, CG)], sem).wait()

    g_start(0, sr0, dr0, gsem0)
    g_start(1, sr1, dr1, gsem1)
    g_wait(sr0, dr0, gsem0)
    w_start(0, sr0, dr0, wsem0)
    g_wait(sr1, dr1, gsem1)
    w_start(1, sr1, dr1, wsem1)

    def body(k, carry):
        c = 2 * k + 2
        w_wait(sr0, dr0, wsem0)
        g_start(c, sr0, dr0, gsem0)
        w_wait(sr1, dr1, wsem1)
        g_start(c + 1, sr1, dr1, gsem1)
        g_wait(sr0, dr0, gsem0)
        w_start(c, sr0, dr0, wsem0)
        g_wait(sr1, dr1, gsem1)
        w_start(c + 1, sr1, dr1, wsem1)
        return carry

    lax.fori_loop(0, (nch - 2) // 2, body, 0)
    if nch % 2:
        w_wait(sr0, dr0, wsem0)
        g_start(nch - 1, sr0, dr0, gsem0)
        g_wait(sr0, dr0, gsem0)
        w_start(nch - 1, sr0, dr0, wsem0)
    w_wait(sr0, dr0, wsem0)
    w_wait(sr1, dr1, wsem1)


def _edge_kernel(gsrc_ref, gdst_ref, bp1_ref, Wp1_ref, Wp2_ref, bp2_ref,
                 wq_ref, cq_ref, Wa2_ref, bb2_ref, sv_ref):
    gsrc = gsrc_ref[...]
    gdst = gdst_ref[...]
    pdiff = gdst[:, :PW] - gsrc[:, :PW]
    bdiff = gdst[:, PW:PW + AH] - gsrc[:, PW:PW + AH]
    xls = gsrc[:, PW + AH:PW + AH + D]
    cq = cq_ref[...]
    t = jax.nn.relu(jnp.dot(pdiff, Wp1_ref[...],
                            preferred_element_type=jnp.float32) + bp1_ref[...])
    delta = jnp.dot(t, Wp2_ref[...], preferred_element_type=jnp.float32) \
        + bp2_ref[...]
    u = bdiff + jnp.dot(t, wq_ref[...], preferred_element_type=jnp.float32) \
        + cq[:, :AH]
    alpha = jnp.dot(jax.nn.relu(u), Wa2_ref[...],
                    preferred_element_type=jnp.float32) + bb2_ref[...]
    s = jnp.exp(alpha)
    sv_ref[0] = s
    sv_ref[1] = s * (xls + delta)


def _sc_scatter_body(dsti_hbm, sv_hbm, zeros_hbm, out_hbm,
                     idx0a, idx0b, idx1a, idx1b, rows0, rows1,
                     lsem0, lsem1, ssem0, ssem1, table_sh):
    c = lax.axis_index("c")
    s = lax.axis_index("s")

    @pl.when(s == 0)
    def _():
        pltpu.sync_copy(zeros_hbm, table_sh)

    ept = dsti_hbm.shape[0] // NS
    nch = ept // CL
    base0 = s * ept
    plsc.subcore_barrier()

    def l_start(ch, idxa, idxb, rows, sem):
        base = base0 + pl.multiple_of(ch * CL, 8)
        pltpu.async_copy(dsti_hbm.at[pl.ds(base, CS)], idxa, sem)
        pltpu.async_copy(dsti_hbm.at[pl.ds(base + CS, CS)], idxb, sem)
        pltpu.async_copy(sv_hbm.at[c, pl.ds(base, CL)], rows, sem)

    def l_wait(idxa, idxb, rows, sem):
        pltpu.make_async_copy(dsti_hbm.at[pl.ds(0, CS)], idxa, sem).wait()
        pltpu.make_async_copy(dsti_hbm.at[pl.ds(0, CS)], idxb, sem).wait()
        pltpu.make_async_copy(sv_hbm.at[0, pl.ds(0, CL)], rows, sem).wait()

    def s_start(idxa, idxb, rows, sem):
        pltpu.async_copy(rows.at[pl.ds(0, CS)], table_sh.at[idxa], sem,
                         add=True)
        pltpu.async_copy(rows.at[pl.ds(CS, CS)], table_sh.at[idxb], sem,
                         add=True)

    def s_wait(rows, sem):
        pltpu.make_async_copy(rows, table_sh.at[pl.ds(0, CL)], sem).wait()

    l_start(0, idx0a, idx0b, rows0, lsem0)
    l_start(1, idx1a, idx1b, rows1, lsem1)
    l_wait(idx0a, idx0b, rows0, lsem0)
    s_start(idx0a, idx0b, rows0, ssem0)
    l_wait(idx1a, idx1b, rows1, lsem1)
    s_start(idx1a, idx1b, rows1, ssem1)

    def body(k, carry):
        ch = 2 * k + 2
        s_wait(rows0, ssem0)
        l_start(ch, idx0a, idx0b, rows0, lsem0)
        s_wait(rows1, ssem1)
        l_start(ch + 1, idx1a, idx1b, rows1, lsem1)
        l_wait(idx0a, idx0b, rows0, lsem0)
        s_start(idx0a, idx0b, rows0, ssem0)
        l_wait(idx1a, idx1b, rows1, lsem1)
        s_start(idx1a, idx1b, rows1, ssem1)
        return carry

    lax.fori_loop(0, (nch - 2) // 2, body, 0)
    if nch % 2:
        s_wait(rows0, ssem0)
        l_start(nch - 1, idx0a, idx0b, rows0, lsem0)
        l_wait(idx0a, idx0b, rows0, lsem0)
        s_start(idx0a, idx0b, rows0, ssem0)
    s_wait(rows0, ssem0)
    s_wait(rows1, ssem1)
    plsc.subcore_barrier()

    rpt = 624
    pltpu.sync_copy(table_sh.at[pl.ds(s * rpt, rpt)],
                    out_hbm.at[c, pl.ds(s * rpt, rpt)])

    @pl.when(s == 0)
    def _():
        pltpu.sync_copy(table_sh.at[pl.ds(NS * rpt, N - NS * rpt)],
                        out_hbm.at[c, pl.ds(NS * rpt, N - NS * rpt)])


def _final_kernel(pd0_ref, pn0_ref, pd1_ref, pn1_ref, s0_ref, v0_ref,
                  W_out_ref, b_out_ref, out_ref):
    denom = pd0_ref[0] + pd1_ref[0] + s0_ref[...]
    num = pn0_ref[0] + pn1_ref[0] + v0_ref[...]
    y = num / (denom + 1e-16)
    out_ref[...] = jax.nn.relu(
        jnp.dot(y, W_out_ref[...], preferred_element_type=jnp.float32)
        + b_out_ref[...])


def kernel(x, pos, edge_index, W_in, b_in, W_lin, W_src, W_dst, Wp1, bp1,
           Wp2, bp2, Wa1, ba1, Wa2, bb2, W_out, b_out):
    f32 = jnp.float32
    pos16 = jnp.pad(pos, ((0, 0), (0, PW - 3)))
    Wp1_16 = jnp.pad(Wp1, ((0, PW - 3), (0, 0)))
    src = edge_index[0].astype(jnp.int32)
    dst = edge_index[1].astype(jnp.int32)
    b_in2 = b_in.reshape(1, D)
    bp1_2 = bp1.reshape(1, AH)
    bp2_2 = bp2.reshape(1, D)
    ba1_2 = ba1.reshape(1, AH)
    bb2_2 = bb2.reshape(1, D)
    b_out2 = b_out.reshape(1, D)

    nsteps = N // RB
    full = lambda shp: pl.BlockSpec(shp, lambda i: tuple(0 for _ in shp))
    rows = lambda w: pl.BlockSpec((RB, w), lambda i: (i, 0))

    tsrc, tdst, s0, v0, wq, cq = pl.pallas_call(
        _node_kernel,
        grid=(nsteps,),
        in_specs=[rows(D), rows(PW), full((D, D)), full((1, D)),
                  full((D, D)), full((D, D)), full((D, D)), full((1, AH)),
                  full((AH, D)), full((1, D)), full((D, AH)), full((1, AH)),
                  full((AH, D)), full((1, D))],
        out_specs=[rows(DS), rows(DD), rows(D), rows(D),
                   full((AH, AH)), full((1, AH + D))],
        out_shape=[jax.ShapeDtypeStruct((N, DS), f32),
                   jax.ShapeDtypeStruct((N, DD), f32),
                   jax.ShapeDtypeStruct((N, D), f32),
                   jax.ShapeDtypeStruct((N, D), f32),
                   jax.ShapeDtypeStruct((AH, AH), f32),
                   jax.ShapeDtypeStruct((1, AH + D), f32)],
    )(x, pos16, W_in, b_in2, W_lin, W_src, W_dst, bp1_2, Wp2, bp2_2,
      Wa1, ba1_2, Wa2, bb2_2)

    mesh = plsc.VectorSubcoreMesh(core_axis_name="c", subcore_axis_name="s")
    H = E // NH
    zeros = jnp.zeros((N, D), f32)
    erows = lambda w: pl.BlockSpec((EB, w), lambda i: (i, 0))

    def gather_half(src_h, dst_h):
        return pl.kernel(
            _sc_gather_body,
            out_type=[jax.ShapeDtypeStruct((H, DS), f32),
                      jax.ShapeDtypeStruct((H, DD), f32)],
            mesh=mesh,
            scratch_types=[pltpu.VMEM((H // NW,), jnp.int32),
                           pltpu.VMEM((H // NW,), jnp.int32),
                           pltpu.VMEM((CG, DS), f32),
                           pltpu.VMEM((CG, DS), f32),
                           pltpu.VMEM((CG, DD), f32),
                           pltpu.VMEM((CG, DD), f32),
                           pltpu.SemaphoreType.DMA,
                           pltpu.SemaphoreType.DMA,
                           pltpu.SemaphoreType.DMA,
                           pltpu.SemaphoreType.DMA],
        )(src_h, dst_h, tsrc, tdst)

    def edge_half(gsrc, gdst):
        return pl.pallas_call(
            _edge_kernel,
            grid=(H // EB,),
            in_specs=[erows(DS), erows(DD), full((1, AH)), full((PW, AH)),
                      full((AH, D)), full((1, D)), full((AH, AH)),
                      full((1, AH + D)), full((AH, D)), full((1, D))],
            out_specs=[pl.BlockSpec((2, EB, D), lambda i: (0, i, 0))],
            out_shape=[jax.ShapeDtypeStruct((2, H, D), f32)],
        )(gsrc, gdst, bp1_2, Wp1_16, Wp2, bp2_2, wq, cq, Wa2, bb2_2)[0]

    def scatter_half(dst_h, sv):
        return pl.kernel(
            _sc_scatter_body,
            out_type=jax.ShapeDtypeStruct((2, N, D), f32),
            mesh=mesh,
            scratch_types=[pltpu.VMEM((CS,), jnp.int32),
                           pltpu.VMEM((CS,), jnp.int32),
                           pltpu.VMEM((CS,), jnp.int32),
                           pltpu.VMEM((CS,), jnp.int32),
                           pltpu.VMEM((CL, D), f32),
                           pltpu.VMEM((CL, D), f32),
                           pltpu.SemaphoreType.DMA,
                           pltpu.SemaphoreType.DMA,
                           pltpu.SemaphoreType.DMA,
                           pltpu.SemaphoreType.DMA,
                           pltpu.VMEM_SHARED((N, D), f32)],
        )(dst_h, sv, zeros)

    srcs = [lax.slice(src, (h * H,), ((h + 1) * H,)) for h in range(NH)]
    dsts = [lax.slice(dst, (h * H,), ((h + 1) * H,)) for h in range(NH)]
    gs = [gather_half(srcs[h], dsts[h]) for h in range(NH)]
    svs = [edge_half(*gs[h]) for h in range(NH)]
    parts = [scatter_half(dsts[h], svs[h]) for h in range(NH)]

    out = pl.pallas_call(
        _final_kernel,
        grid=(nsteps,),
        in_specs=[pl.BlockSpec((1, RB, D), lambda i: (0, i, 0)),
                  pl.BlockSpec((1, RB, D), lambda i: (1, i, 0)),
                  pl.BlockSpec((1, RB, D), lambda i: (0, i, 0)),
                  pl.BlockSpec((1, RB, D), lambda i: (1, i, 0)),
                  rows(D), rows(D), full((D, D)), full((1, D))],
        out_specs=[rows(D)],
        out_shape=[jax.ShapeDtypeStruct((N, D), f32)],
    )(parts[0], parts[0], parts[1], parts[1], s0, v0, W_out, b_out2)[0]
    return out


# speedup vs baseline: 10.7795x; 1.0489x over previous
import functools

import jax
import jax.numpy as jnp
from jax import lax
from jax.experimental import pallas as pl
from jax.experimental.pallas import tpu as pltpu
from jax.experimental.pallas import tpu_sc as plsc

N = 10000
E = 320000
D = 128
AH = 64
PW = 16
DS = 256
DD = 128

NC = 2
NS = 16
NW = NC * NS

NH = 2
NB = 4
CG = 40
CS = 40
CL = 2 * CS

RB = 2000
EB = 1280


def _node_kernel(x_ref, pos_ref, W_in_ref, b_in_ref, W_lin_ref, W_src_ref,
                 W_dst_ref, bp1_ref, Wp2_ref, bp2_ref, Wa1_ref, ba1_ref,
                 Wa2_ref, bb2_ref,
                 tsrc_ref, tdst_ref, s0_ref, v0_ref, wq_ref, cq_ref):
    x = x_ref[...]
    pos16 = pos_ref[...]
    h = jax.nn.relu(jnp.dot(x, W_in_ref[...], preferred_element_type=jnp.float32)
                    + b_in_ref[...])
    a_src = jnp.dot(h, W_src_ref[...], preferred_element_type=jnp.float32)
    a_dst = jnp.dot(h, W_dst_ref[...], preferred_element_type=jnp.float32)
    xl = jnp.dot(h, W_lin_ref[...], preferred_element_type=jnp.float32)
    Wa1 = Wa1_ref[...]
    bsrc = jnp.dot(a_src, Wa1, preferred_element_type=jnp.float32)
    bdst = jnp.dot(a_dst, Wa1, preferred_element_type=jnp.float32)
    zpad = jnp.zeros((pos16.shape[0], DS - PW - AH - D), dtype=jnp.float32)
    tsrc_ref[...] = jnp.concatenate([pos16, bsrc, xl, zpad], axis=1)
    tdst_ref[...] = jnp.concatenate([pos16, bdst, zpad], axis=1)
    Wp2 = Wp2_ref[...]
    wq_ref[...] = jnp.dot(Wp2, Wa1, preferred_element_type=jnp.float32)
    d0 = jnp.dot(jax.nn.relu(bp1_ref[...]), Wp2,
                 preferred_element_type=jnp.float32) + bp2_ref[...]
    cq_ref[...] = jnp.concatenate(
        [jnp.dot(bp2_ref[...], Wa1, preferred_element_type=jnp.float32)
         + ba1_ref[...], d0], axis=1)
    u0 = bdst - bsrc + jnp.dot(d0, Wa1, preferred_element_type=jnp.float32) \
        + ba1_ref[...]
    alpha0 = jnp.dot(jax.nn.relu(u0), Wa2_ref[...],
                     preferred_element_type=jnp.float32) + bb2_ref[...]
    s0 = jnp.exp(alpha0)
    s0_ref[...] = s0
    v0_ref[...] = s0 * (xl + d0)


def _sc_gather_body(src_hbm, dst_hbm, tsrc_hbm, tdst_hbm,
                    gsrc_out, gdst_out, sidx, didx, *slots):
    NB = len(slots) // 4
    srs, drs, gsems, wsems = (slots[i * NB:(i + 1) * NB] for i in range(4))
    wid = lax.axis_index("s") * NC + lax.axis_index("c")
    epw = src_hbm.shape[0] // NW
    nch = epw // CG
    base0 = wid * epw
    pltpu.sync_copy(src_hbm.at[pl.ds(base0, epw)], sidx)
    pltpu.sync_copy(dst_hbm.at[pl.ds(base0, epw)], didx)

    def g_start(c, b):
        off = pl.multiple_of(c * CG, 8)
        pltpu.async_copy(tsrc_hbm.at[sidx.at[pl.ds(off, CG)]], srs[b],
                         gsems[b])
        pltpu.async_copy(tdst_hbm.at[didx.at[pl.ds(off, CG)]], drs[b],
                         gsems[b])

    def g_wait(b):
        pltpu.make_async_copy(tsrc_hbm.at[pl.ds(0, CG)], srs[b],
                              gsems[b]).wait()
        pltpu.make_async_copy(tdst_hbm.at[pl.ds(0, CG)], drs[b],
                              gsems[b]).wait()

    def w_start(c, b):
        base = base0 + pl.multiple_of(c * CG, 8)
        pltpu.async_copy(srs[b], gsrc_out.at[pl.ds(base, CG)], wsems[b])
        pltpu.async_copy(drs[b], gdst_out.at[pl.ds(base, CG)], wsems[b])

    def w_wait(b):
        pltpu.make_async_copy(srs[b], gsrc_out.at[pl.ds(0, CG)],
                              wsems[b]).wait()
        pltpu.make_async_copy(drs[b], gdst_out.at[pl.ds(0, CG)],
                              wsems[b]).wait()

    for b in range(NB):
        g_start(b, b)

    def body(k, carry):
        c = k * NB
        for b in range(NB):
            g_wait(b)
            w_start(c + b, b)
        for b in range(NB):
            w_wait(b)

            @pl.when(c + NB + b < nch - 1)
            def _(b=b, nc=c + NB + b):
                g_start(nc, b)
        return carry

    lax.fori_loop(0, (nch - 1) // NB, body, 0)
    g_start(nch - 1, 0)
    g_wait(0)
    w_start(nch - 1, 0)
    w_wait(0)


def _edge_kernel(gsrc_ref, gdst_ref, bp1_ref, Wp1_ref, Wp2_ref, bp2_ref,
                 wq_ref, cq_ref, Wa2_ref, bb2_ref, sv_ref):
    gsrc = gsrc_ref[...]
    gdst = gdst_ref[...]
    pdiff = gdst[:, :PW] - gsrc[:, :PW]
    bdiff = gdst[:, PW:PW + AH] - gsrc[:, PW:PW + AH]
    xls = gsrc[:, PW + AH:PW + AH + D]
    cq = cq_ref[...]
    t = jax.nn.relu(jnp.dot(pdiff, Wp1_ref[...],
                            preferred_element_type=jnp.float32) + bp1_ref[...])
    delta = jnp.dot(t, Wp2_ref[...], preferred_element_type=jnp.float32) \
        + bp2_ref[...]
    u = bdiff + jnp.dot(t, wq_ref[...], preferred_element_type=jnp.float32) \
        + cq[:, :AH]
    alpha = jnp.dot(jax.nn.relu(u), Wa2_ref[...],
                    preferred_element_type=jnp.float32) + bb2_ref[...]
    s = jnp.exp(alpha)
    sv_ref[0] = s
    sv_ref[1] = s * (xls + delta)


def _sc_scatter_body(dsti_hbm, sv_hbm, zeros_hbm, out_hbm, *slots):
    table_sh = slots[-1]
    slots = slots[:-1]
    NB = len(slots) // 5
    idxa, idxb, rowss, lsems, ssems = (slots[i * NB:(i + 1) * NB]
                                       for i in range(5))
    c = lax.axis_index("c")
    s = lax.axis_index("s")

    @pl.when(s == 0)
    def _():
        pltpu.sync_copy(zeros_hbm, table_sh)

    ept = dsti_hbm.shape[0] // NS
    nch = ept // CL
    base0 = s * ept
    plsc.subcore_barrier()

    def l_start(ch, b):
        base = base0 + pl.multiple_of(ch * CL, 8)
        pltpu.async_copy(dsti_hbm.at[pl.ds(base, CS)], idxa[b], lsems[b])
        pltpu.async_copy(dsti_hbm.at[pl.ds(base + CS, CS)], idxb[b], lsems[b])
        pltpu.async_copy(sv_hbm.at[c, pl.ds(base, CL)], rowss[b], lsems[b])

    def l_wait(b):
        pltpu.make_async_copy(dsti_hbm.at[pl.ds(0, CS)], idxa[b],
                              lsems[b]).wait()
        pltpu.make_async_copy(dsti_hbm.at[pl.ds(0, CS)], idxb[b],
                              lsems[b]).wait()
        pltpu.make_async_copy(sv_hbm.at[0, pl.ds(0, CL)], rowss[b],
                              lsems[b]).wait()

    def s_start(b):
        pltpu.async_copy(rowss[b].at[pl.ds(0, CS)], table_sh.at[idxa[b]],
                         ssems[b], add=True)
        pltpu.async_copy(rowss[b].at[pl.ds(CS, CS)], table_sh.at[idxb[b]],
                         ssems[b], add=True)

    def s_wait(b):
        pltpu.make_async_copy(rowss[b], table_sh.at[pl.ds(0, CL)],
                              ssems[b]).wait()

    for b in range(NB):
        l_start(b, b)

    def body(k, carry):
        ch = k * NB
        for b in range(NB):
            l_wait(b)
            s_start(b)
        for b in range(NB):
            s_wait(b)

            @pl.when(ch + NB + b < nch - 1)
            def _(b=b, nc=ch + NB + b):
                l_start(nc, b)
        return carry

    lax.fori_loop(0, (nch - 1) // NB, body, 0)
    l_start(nch - 1, 0)
    l_wait(0)
    s_start(0)
    s_wait(0)
    plsc.subcore_barrier()

    rpt = 624
    pltpu.sync_copy(table_sh.at[pl.ds(s * rpt, rpt)],
                    out_hbm.at[c, pl.ds(s * rpt, rpt)])

    @pl.when(s == 0)
    def _():
        pltpu.sync_copy(table_sh.at[pl.ds(NS * rpt, N - NS * rpt)],
                        out_hbm.at[c, pl.ds(NS * rpt, N - NS * rpt)])


def _final_kernel(pd0_ref, pn0_ref, pd1_ref, pn1_ref, s0_ref, v0_ref,
                  W_out_ref, b_out_ref, out_ref):
    denom = pd0_ref[0] + pd1_ref[0] + s0_ref[...]
    num = pn0_ref[0] + pn1_ref[0] + v0_ref[...]
    y = num / (denom + 1e-16)
    out_ref[...] = jax.nn.relu(
        jnp.dot(y, W_out_ref[...], preferred_element_type=jnp.float32)
        + b_out_ref[...])


def kernel(x, pos, edge_index, W_in, b_in, W_lin, W_src, W_dst, Wp1, bp1,
           Wp2, bp2, Wa1, ba1, Wa2, bb2, W_out, b_out):
    f32 = jnp.float32
    pos16 = jnp.pad(pos, ((0, 0), (0, PW - 3)))
    Wp1_16 = jnp.pad(Wp1, ((0, PW - 3), (0, 0)))
    src = edge_index[0].astype(jnp.int32)
    dst = edge_index[1].astype(jnp.int32)
    b_in2 = b_in.reshape(1, D)
    bp1_2 = bp1.reshape(1, AH)
    bp2_2 = bp2.reshape(1, D)
    ba1_2 = ba1.reshape(1, AH)
    bb2_2 = bb2.reshape(1, D)
    b_out2 = b_out.reshape(1, D)

    nsteps = N // RB
    full = lambda shp: pl.BlockSpec(shp, lambda i: tuple(0 for _ in shp))
    rows = lambda w: pl.BlockSpec((RB, w), lambda i: (i, 0))

    tsrc, tdst, s0, v0, wq, cq = pl.pallas_call(
        _node_kernel,
        grid=(nsteps,),
        in_specs=[rows(D), rows(PW), full((D, D)), full((1, D)),
                  full((D, D)), full((D, D)), full((D, D)), full((1, AH)),
                  full((AH, D)), full((1, D)), full((D, AH)), full((1, AH)),
                  full((AH, D)), full((1, D))],
        out_specs=[rows(DS), rows(DD), rows(D), rows(D),
                   full((AH, AH)), full((1, AH + D))],
        out_shape=[jax.ShapeDtypeStruct((N, DS), f32),
                   jax.ShapeDtypeStruct((N, DD), f32),
                   jax.ShapeDtypeStruct((N, D), f32),
                   jax.ShapeDtypeStruct((N, D), f32),
                   jax.ShapeDtypeStruct((AH, AH), f32),
                   jax.ShapeDtypeStruct((1, AH + D), f32)],
    )(x, pos16, W_in, b_in2, W_lin, W_src, W_dst, bp1_2, Wp2, bp2_2,
      Wa1, ba1_2, Wa2, bb2_2)

    mesh = plsc.VectorSubcoreMesh(core_axis_name="c", subcore_axis_name="s")
    H = E // NH
    zeros = jnp.zeros((N, D), f32)
    erows = lambda w: pl.BlockSpec((EB, w), lambda i: (i, 0))

    def gather_half(src_h, dst_h):
        return pl.kernel(
            _sc_gather_body,
            out_type=[jax.ShapeDtypeStruct((H, DS), f32),
                      jax.ShapeDtypeStruct((H, DD), f32)],
            mesh=mesh,
            scratch_types=([pltpu.VMEM((H // NW,), jnp.int32)] * 2
                           + [pltpu.VMEM((CG, DS), f32)] * NB
                           + [pltpu.VMEM((CG, DD), f32)] * NB
                           + [pltpu.SemaphoreType.DMA] * (2 * NB)),
        )(src_h, dst_h, tsrc, tdst)

    def edge_half(gsrc, gdst):
        return pl.pallas_call(
            _edge_kernel,
            grid=(H // EB,),
            in_specs=[erows(DS), erows(DD), full((1, AH)), full((PW, AH)),
                      full((AH, D)), full((1, D)), full((AH, AH)),
                      full((1, AH + D)), full((AH, D)), full((1, D))],
            out_specs=[pl.BlockSpec((2, EB, D), lambda i: (0, i, 0))],
            out_shape=[jax.ShapeDtypeStruct((2, H, D), f32)],
        )(gsrc, gdst, bp1_2, Wp1_16, Wp2, bp2_2, wq, cq, Wa2, bb2_2)[0]

    def scatter_half(dst_h, sv):
        return pl.kernel(
            _sc_scatter_body,
            out_type=jax.ShapeDtypeStruct((2, N, D), f32),
            mesh=mesh,
            scratch_types=([pltpu.VMEM((CS,), jnp.int32)] * (2 * NB)
                           + [pltpu.VMEM((CL, D), f32)] * NB
                           + [pltpu.SemaphoreType.DMA] * (2 * NB)
                           + [pltpu.VMEM_SHARED((N, D), f32)]),
        )(dst_h, sv, zeros)

    srcs = [lax.slice(src, (h * H,), ((h + 1) * H,)) for h in range(NH)]
    dsts = [lax.slice(dst, (h * H,), ((h + 1) * H,)) for h in range(NH)]
    gs = [gather_half(srcs[h], dsts[h]) for h in range(NH)]
    svs = [edge_half(*gs[h]) for h in range(NH)]
    parts = [scatter_half(dsts[h], svs[h]) for h in range(NH)]

    out = pl.pallas_call(
        _final_kernel,
        grid=(nsteps,),
        in_specs=[pl.BlockSpec((1, RB, D), lambda i: (0, i, 0)),
                  pl.BlockSpec((1, RB, D), lambda i: (1, i, 0)),
                  pl.BlockSpec((1, RB, D), lambda i: (0, i, 0)),
                  pl.BlockSpec((1, RB, D), lambda i: (1, i, 0)),
                  rows(D), rows(D), full((D, D)), full((1, D))],
        out_specs=[rows(D)],
        out_shape=[jax.ShapeDtypeStruct((N, D), f32)],
    )(parts[0], parts[0], parts[1], parts[1], s0, v0, W_out, b_out2)[0]
    return out


# speedup vs baseline: 13.0741x; 1.2129x over previous
import functools

import jax
import jax.numpy as jnp
from jax import lax
from jax.experimental import pallas as pl
from jax.experimental.pallas import tpu as pltpu
from jax.experimental.pallas import tpu_sc as plsc

N = 10000
E = 320000
D = 128
AH = 64
PW = 16
DS = 128
DD = 128

NC = 2
NS = 16
NW = NC * NS

NH = 2
NB = 4
CG = 40
CS = 40
CL = 2 * CS

RB = 2000
EB = 1280


def _node_kernel(x_ref, pos_ref, W_in_ref, b_in_ref, W_lin_ref, W_src_ref,
                 W_dst_ref, bp1_ref, Wp2_ref, bp2_ref, Wa1_ref, ba1_ref,
                 Wa2_ref, bb2_ref,
                 tsrc_ref, tdst_ref, s0_ref, v0_ref, wq_ref, cq_ref):
    x = x_ref[...]
    pos16 = pos_ref[...]
    h = jax.nn.relu(jnp.dot(x, W_in_ref[...], preferred_element_type=jnp.float32)
                    + b_in_ref[...])
    a_src = jnp.dot(h, W_src_ref[...], preferred_element_type=jnp.float32)
    a_dst = jnp.dot(h, W_dst_ref[...], preferred_element_type=jnp.float32)
    xl = jnp.dot(h, W_lin_ref[...], preferred_element_type=jnp.float32)
    Wa1 = Wa1_ref[...]
    bsrc = jnp.dot(a_src, Wa1, preferred_element_type=jnp.float32)
    bdst = jnp.dot(a_dst, Wa1, preferred_element_type=jnp.float32)
    zpad = jnp.zeros((pos16.shape[0], DD - PW - AH), dtype=jnp.float32)
    hi = jnp.concatenate([pos16, bsrc, zpad], axis=1)
    uhi = lax.bitcast_convert_type(hi, jnp.uint32)
    ulo = lax.bitcast_convert_type(xl, jnp.uint32)
    packed = ((uhi + 0x8000) & jnp.uint32(0xFFFF0000)) | ((ulo + 0x8000) >> 16)
    tsrc_ref[...] = lax.bitcast_convert_type(packed, jnp.float32)
    tdst_ref[...] = jnp.concatenate([pos16, bdst, zpad], axis=1)
    Wp2 = Wp2_ref[...]
    wq_ref[...] = jnp.dot(Wp2, Wa1, preferred_element_type=jnp.float32)
    d0 = jnp.dot(jax.nn.relu(bp1_ref[...]), Wp2,
                 preferred_element_type=jnp.float32) + bp2_ref[...]
    cq_ref[...] = jnp.concatenate(
        [jnp.dot(bp2_ref[...], Wa1, preferred_element_type=jnp.float32)
         + ba1_ref[...], d0], axis=1)
    u0 = bdst - bsrc + jnp.dot(d0, Wa1, preferred_element_type=jnp.float32) \
        + ba1_ref[...]
    alpha0 = jnp.dot(jax.nn.relu(u0), Wa2_ref[...],
                     preferred_element_type=jnp.float32) + bb2_ref[...]
    s0 = jnp.exp(alpha0)
    s0_ref[...] = s0
    v0_ref[...] = s0 * (xl + d0)


def _sc_gather_body(src_hbm, dst_hbm, tsrc_hbm, tdst_hbm,
                    gsrc_out, gdst_out, sidx, didx, *slots):
    NB = len(slots) // 4
    srs, drs, gsems, wsems = (slots[i * NB:(i + 1) * NB] for i in range(4))
    wid = lax.axis_index("s") * NC + lax.axis_index("c")
    epw = src_hbm.shape[0] // NW
    nch = epw // CG
    base0 = wid * epw
    pltpu.sync_copy(src_hbm.at[pl.ds(base0, epw)], sidx)
    pltpu.sync_copy(dst_hbm.at[pl.ds(base0, epw)], didx)

    def g_start(c, b):
        off = pl.multiple_of(c * CG, 8)
        pltpu.async_copy(tsrc_hbm.at[sidx.at[pl.ds(off, CG)]], srs[b],
                         gsems[b])
        pltpu.async_copy(tdst_hbm.at[didx.at[pl.ds(off, CG)]], drs[b],
                         gsems[b])

    def g_wait(b):
        pltpu.make_async_copy(tsrc_hbm.at[pl.ds(0, CG)], srs[b],
                              gsems[b]).wait()
        pltpu.make_async_copy(tdst_hbm.at[pl.ds(0, CG)], drs[b],
                              gsems[b]).wait()

    def w_start(c, b):
        base = base0 + pl.multiple_of(c * CG, 8)
        pltpu.async_copy(srs[b], gsrc_out.at[pl.ds(base, CG)], wsems[b])
        pltpu.async_copy(drs[b], gdst_out.at[pl.ds(base, CG)], wsems[b])

    def w_wait(b):
        pltpu.make_async_copy(srs[b], gsrc_out.at[pl.ds(0, CG)],
                              wsems[b]).wait()
        pltpu.make_async_copy(drs[b], gdst_out.at[pl.ds(0, CG)],
                              wsems[b]).wait()

    for b in range(NB):
        g_start(b, b)

    def body(k, carry):
        c = k * NB
        for b in range(NB):
            g_wait(b)
            w_start(c + b, b)
        for b in range(NB):
            w_wait(b)

            @pl.when(c + NB + b < nch - 1)
            def _(b=b, nc=c + NB + b):
                g_start(nc, b)
        return carry

    lax.fori_loop(0, (nch - 1) // NB, body, 0)
    g_start(nch - 1, 0)
    g_wait(0)
    w_start(nch - 1, 0)
    w_wait(0)


def _edge_kernel(gsrc_ref, gdst_ref, bp1_ref, Wp1_ref, Wp2_ref, bp2_ref,
                 wq_ref, cq_ref, Wa2_ref, bb2_ref, sv_ref):
    us = lax.bitcast_convert_type(gsrc_ref[...], jnp.uint32)
    his = lax.bitcast_convert_type(us & jnp.uint32(0xFFFF0000), jnp.float32)
    xls = lax.bitcast_convert_type(us << 16, jnp.float32)
    gdst = gdst_ref[...]
    pdiff = gdst[:, :PW] - his[:, :PW]
    bdiff = gdst[:, PW:PW + AH] - his[:, PW:PW + AH]
    cq = cq_ref[...]
    t = jax.nn.relu(jnp.dot(pdiff, Wp1_ref[...],
                            preferred_element_type=jnp.float32) + bp1_ref[...])
    delta = jnp.dot(t, Wp2_ref[...], preferred_element_type=jnp.float32) \
        + bp2_ref[...]
    u = bdiff + jnp.dot(t, wq_ref[...], preferred_element_type=jnp.float32) \
        + cq[:, :AH]
    alpha = jnp.dot(jax.nn.relu(u), Wa2_ref[...],
                    preferred_element_type=jnp.float32) + bb2_ref[...]
    s = jnp.exp(alpha)
    sv_ref[0] = s
    sv_ref[1] = s * (xls + delta)


def _sc_scatter_body(dsti_hbm, sv_hbm, zeros_hbm, out_hbm, *slots):
    table_sh = slots[-1]
    slots = slots[:-1]
    NB = len(slots) // 5
    idxa, idxb, rowss, lsems, ssems = (slots[i * NB:(i + 1) * NB]
                                       for i in range(5))
    c = lax.axis_index("c")
    s = lax.axis_index("s")

    @pl.when(s == 0)
    def _():
        pltpu.sync_copy(zeros_hbm, table_sh)

    ept = dsti_hbm.shape[0] // NS
    nch = ept // CL
    base0 = s * ept
    plsc.subcore_barrier()

    def l_start(ch, b):
        base = base0 + pl.multiple_of(ch * CL, 8)
        pltpu.async_copy(dsti_hbm.at[pl.ds(base, CS)], idxa[b], lsems[b])
        pltpu.async_copy(dsti_hbm.at[pl.ds(base + CS, CS)], idxb[b], lsems[b])
        pltpu.async_copy(sv_hbm.at[c, pl.ds(base, CL)], rowss[b], lsems[b])

    def l_wait(b):
        pltpu.make_async_copy(dsti_hbm.at[pl.ds(0, CS)], idxa[b],
                              lsems[b]).wait()
        pltpu.make_async_copy(dsti_hbm.at[pl.ds(0, CS)], idxb[b],
                              lsems[b]).wait()
        pltpu.make_async_copy(sv_hbm.at[0, pl.ds(0, CL)], rowss[b],
                              lsems[b]).wait()

    def s_start(b):
        pltpu.async_copy(rowss[b].at[pl.ds(0, CS)], table_sh.at[idxa[b]],
                         ssems[b], add=True)
        pltpu.async_copy(rowss[b].at[pl.ds(CS, CS)], table_sh.at[idxb[b]],
                         ssems[b], add=True)

    def s_wait(b):
        pltpu.make_async_copy(rowss[b], table_sh.at[pl.ds(0, CL)],
                              ssems[b]).wait()

    for b in range(NB):
        l_start(b, b)

    def body(k, carry):
        ch = k * NB
        for b in range(NB):
            l_wait(b)
            s_start(b)
        for b in range(NB):
            s_wait(b)

            @pl.when(ch + NB + b < nch - 1)
            def _(b=b, nc=ch + NB + b):
                l_start(nc, b)
        return carry

    lax.fori_loop(0, (nch - 1) // NB, body, 0)
    l_start(nch - 1, 0)
    l_wait(0)
    s_start(0)
    s_wait(0)
    plsc.subcore_barrier()

    rpt = 624
    pltpu.sync_copy(table_sh.at[pl.ds(s * rpt, rpt)],
                    out_hbm.at[c, pl.ds(s * rpt, rpt)])

    @pl.when(s == 0)
    def _():
        pltpu.sync_copy(table_sh.at[pl.ds(NS * rpt, N - NS * rpt)],
                        out_hbm.at[c, pl.ds(NS * rpt, N - NS * rpt)])


def _final_kernel(pd0_ref, pn0_ref, pd1_ref, pn1_ref, s0_ref, v0_ref,
                  W_out_ref, b_out_ref, out_ref):
    denom = pd0_ref[0] + pd1_ref[0] + s0_ref[...]
    num = pn0_ref[0] + pn1_ref[0] + v0_ref[...]
    y = num / (denom + 1e-16)
    out_ref[...] = jax.nn.relu(
        jnp.dot(y, W_out_ref[...], preferred_element_type=jnp.float32)
        + b_out_ref[...])


def kernel(x, pos, edge_index, W_in, b_in, W_lin, W_src, W_dst, Wp1, bp1,
           Wp2, bp2, Wa1, ba1, Wa2, bb2, W_out, b_out):
    f32 = jnp.float32
    pos16 = jnp.pad(pos, ((0, 0), (0, PW - 3)))
    Wp1_16 = jnp.pad(Wp1, ((0, PW - 3), (0, 0)))
    src = edge_index[0].astype(jnp.int32)
    dst = edge_index[1].astype(jnp.int32)
    b_in2 = b_in.reshape(1, D)
    bp1_2 = bp1.reshape(1, AH)
    bp2_2 = bp2.reshape(1, D)
    ba1_2 = ba1.reshape(1, AH)
    bb2_2 = bb2.reshape(1, D)
    b_out2 = b_out.reshape(1, D)

    nsteps = N // RB
    full = lambda shp: pl.BlockSpec(shp, lambda i: tuple(0 for _ in shp))
    rows = lambda w: pl.BlockSpec((RB, w), lambda i: (i, 0))

    tsrc, tdst, s0, v0, wq, cq = pl.pallas_call(
        _node_kernel,
        grid=(nsteps,),
        in_specs=[rows(D), rows(PW), full((D, D)), full((1, D)),
                  full((D, D)), full((D, D)), full((D, D)), full((1, AH)),
                  full((AH, D)), full((1, D)), full((D, AH)), full((1, AH)),
                  full((AH, D)), full((1, D))],
        out_specs=[rows(DS), rows(DD), rows(D), rows(D),
                   full((AH, AH)), full((1, AH + D))],
        out_shape=[jax.ShapeDtypeStruct((N, DS), f32),
                   jax.ShapeDtypeStruct((N, DD), f32),
                   jax.ShapeDtypeStruct((N, D), f32),
                   jax.ShapeDtypeStruct((N, D), f32),
                   jax.ShapeDtypeStruct((AH, AH), f32),
                   jax.ShapeDtypeStruct((1, AH + D), f32)],
    )(x, pos16, W_in, b_in2, W_lin, W_src, W_dst, bp1_2, Wp2, bp2_2,
      Wa1, ba1_2, Wa2, bb2_2)

    mesh = plsc.VectorSubcoreMesh(core_axis_name="c", subcore_axis_name="s")
    H = E // NH
    zeros = jnp.zeros((N, D), f32)
    erows = lambda w: pl.BlockSpec((EB, w), lambda i: (i, 0))

    def gather_half(src_h, dst_h):
        return pl.kernel(
            _sc_gather_body,
            out_type=[jax.ShapeDtypeStruct((H, DS), f32),
                      jax.ShapeDtypeStruct((H, DD), f32)],
            mesh=mesh,
            scratch_types=([pltpu.VMEM((H // NW,), jnp.int32)] * 2
                           + [pltpu.VMEM((CG, DS), f32)] * NB
                           + [pltpu.VMEM((CG, DD), f32)] * NB
                           + [pltpu.SemaphoreType.DMA] * (2 * NB)),
        )(src_h, dst_h, tsrc, tdst)

    def edge_half(gsrc, gdst):
        return pl.pallas_call(
            _edge_kernel,
            grid=(H // EB,),
            in_specs=[erows(DS), erows(DD), full((1, AH)), full((PW, AH)),
                      full((AH, D)), full((1, D)), full((AH, AH)),
                      full((1, AH + D)), full((AH, D)), full((1, D))],
            out_specs=[pl.BlockSpec((2, EB, D), lambda i: (0, i, 0))],
            out_shape=[jax.ShapeDtypeStruct((2, H, D), f32)],
        )(gsrc, gdst, bp1_2, Wp1_16, Wp2, bp2_2, wq, cq, Wa2, bb2_2)[0]

    def scatter_half(dst_h, sv):
        return pl.kernel(
            _sc_scatter_body,
            out_type=jax.ShapeDtypeStruct((2, N, D), f32),
            mesh=mesh,
            scratch_types=([pltpu.VMEM((CS,), jnp.int32)] * (2 * NB)
                           + [pltpu.VMEM((CL, D), f32)] * NB
                           + [pltpu.SemaphoreType.DMA] * (2 * NB)
                           + [pltpu.VMEM_SHARED((N, D), f32)]),
        )(dst_h, sv, zeros)

    srcs = [lax.slice(src, (h * H,), ((h + 1) * H,)) for h in range(NH)]
    dsts = [lax.slice(dst, (h * H,), ((h + 1) * H,)) for h in range(NH)]
    gs = [gather_half(srcs[h], dsts[h]) for h in range(NH)]
    svs = [edge_half(*gs[h]) for h in range(NH)]
    parts = [scatter_half(dsts[h], svs[h]) for h in range(NH)]

    out = pl.pallas_call(
        _final_kernel,
        grid=(nsteps,),
        in_specs=[pl.BlockSpec((1, RB, D), lambda i: (0, i, 0)),
                  pl.BlockSpec((1, RB, D), lambda i: (1, i, 0)),
                  pl.BlockSpec((1, RB, D), lambda i: (0, i, 0)),
                  pl.BlockSpec((1, RB, D), lambda i: (1, i, 0)),
                  rows(D), rows(D), full((D, D)), full((1, D))],
        out_specs=[rows(D)],
        out_shape=[jax.ShapeDtypeStruct((N, D), f32)],
    )(parts[0], parts[0], parts[1], parts[1], s0, v0, W_out, b_out2)[0]
    return out


# speedup vs baseline: 14.2437x; 1.0895x over previous
import functools

import jax
import jax.numpy as jnp
from jax import lax
from jax.experimental import pallas as pl
from jax.experimental.pallas import tpu as pltpu
from jax.experimental.pallas import tpu_sc as plsc

N = 10000
E = 320000
D = 128
AH = 64
PW = 16
DS = 128
DD = 128

NC = 2
NS = 16
NW = NC * NS

NH = 2
NB = 4
CG = 40
CS = 40
CL = 2 * CS

RB = 2000
EB = 3200


def _node_kernel(x_ref, pos_ref, W_in_ref, b_in_ref, W_lin_ref, W_src_ref,
                 W_dst_ref, bp1_ref, Wp2_ref, bp2_ref, Wa1_ref, ba1_ref,
                 Wa2_ref, bb2_ref,
                 tsrc_ref, tdst_ref, s0_ref, v0_ref, wq_ref, cq_ref):
    x = x_ref[...]
    pos16 = pos_ref[...]
    h = jax.nn.relu(jnp.dot(x, W_in_ref[...], preferred_element_type=jnp.float32)
                    + b_in_ref[...])
    a_src = jnp.dot(h, W_src_ref[...], preferred_element_type=jnp.float32)
    a_dst = jnp.dot(h, W_dst_ref[...], preferred_element_type=jnp.float32)
    xl = jnp.dot(h, W_lin_ref[...], preferred_element_type=jnp.float32)
    Wa1 = Wa1_ref[...]
    bsrc = jnp.dot(a_src, Wa1, preferred_element_type=jnp.float32)
    bdst = jnp.dot(a_dst, Wa1, preferred_element_type=jnp.float32)
    zpad = jnp.zeros((pos16.shape[0], DD - PW - AH), dtype=jnp.float32)
    hi = jnp.concatenate([pos16, bsrc, zpad], axis=1)
    uhi = lax.bitcast_convert_type(hi, jnp.uint32)
    ulo = lax.bitcast_convert_type(xl, jnp.uint32)
    packed = ((uhi + 0x8000) & jnp.uint32(0xFFFF0000)) | ((ulo + 0x8000) >> 16)
    tsrc_ref[...] = lax.bitcast_convert_type(packed, jnp.float32)
    tdst_ref[...] = jnp.concatenate([pos16, bdst, zpad], axis=1)
    Wp2 = Wp2_ref[...]
    wq_ref[...] = jnp.dot(Wp2, Wa1, preferred_element_type=jnp.float32)
    d0 = jnp.dot(jax.nn.relu(bp1_ref[...]), Wp2,
                 preferred_element_type=jnp.float32) + bp2_ref[...]
    cq_ref[...] = jnp.concatenate(
        [jnp.dot(bp2_ref[...], Wa1, preferred_element_type=jnp.float32)
         + ba1_ref[...], d0], axis=1)
    u0 = bdst - bsrc + jnp.dot(d0, Wa1, preferred_element_type=jnp.float32) \
        + ba1_ref[...]
    alpha0 = jnp.dot(jax.nn.relu(u0), Wa2_ref[...],
                     preferred_element_type=jnp.float32) + bb2_ref[...]
    s0 = jnp.exp(alpha0)
    s0_ref[...] = s0
    v0_ref[...] = s0 * (xl + d0)


def _sc_gather_body(src_hbm, dst_hbm, tsrc_hbm, tdst_hbm,
                    gsrc_out, gdst_out, sidx, didx, *slots):
    NB = len(slots) // 4
    srs, drs, gsems, wsems = (slots[i * NB:(i + 1) * NB] for i in range(4))
    wid = lax.axis_index("s") * NC + lax.axis_index("c")
    epw = src_hbm.shape[0] // NW
    nch = epw // CG
    base0 = wid * epw
    pltpu.sync_copy(src_hbm.at[pl.ds(base0, epw)], sidx)
    pltpu.sync_copy(dst_hbm.at[pl.ds(base0, epw)], didx)

    def g_start(c, b):
        off = pl.multiple_of(c * CG, 8)
        pltpu.async_copy(tsrc_hbm.at[sidx.at[pl.ds(off, CG)]], srs[b],
                         gsems[b])
        pltpu.async_copy(tdst_hbm.at[didx.at[pl.ds(off, CG)]], drs[b],
                         gsems[b])

    def g_wait(b):
        pltpu.make_async_copy(tsrc_hbm.at[pl.ds(0, CG)], srs[b],
                              gsems[b]).wait()
        pltpu.make_async_copy(tdst_hbm.at[pl.ds(0, CG)], drs[b],
                              gsems[b]).wait()

    def w_start(c, b):
        base = base0 + pl.multiple_of(c * CG, 8)
        pltpu.async_copy(srs[b], gsrc_out.at[pl.ds(base, CG)], wsems[b])
        pltpu.async_copy(drs[b], gdst_out.at[pl.ds(base, CG)], wsems[b])

    def w_wait(b):
        pltpu.make_async_copy(srs[b], gsrc_out.at[pl.ds(0, CG)],
                              wsems[b]).wait()
        pltpu.make_async_copy(drs[b], gdst_out.at[pl.ds(0, CG)],
                              wsems[b]).wait()

    for b in range(NB):
        g_start(b, b)

    def body(k, carry):
        c = k * NB
        for b in range(NB):
            g_wait(b)
            w_start(c + b, b)
        for b in range(NB):
            w_wait(b)

            @pl.when(c + NB + b < nch - 1)
            def _(b=b, nc=c + NB + b):
                g_start(nc, b)
        return carry

    lax.fori_loop(0, (nch - 1) // NB, body, 0)
    g_start(nch - 1, 0)
    g_wait(0)
    w_start(nch - 1, 0)
    w_wait(0)


def _edge_kernel(gsrc_ref, gdst_ref, bp1_ref, Wp1_ref, Wp2_ref, bp2_ref,
                 wq_ref, cq_ref, Wa2_ref, bb2_ref, sv_ref):
    us = lax.bitcast_convert_type(gsrc_ref[...], jnp.uint32)
    his = lax.bitcast_convert_type(us & jnp.uint32(0xFFFF0000), jnp.float32)
    xls = lax.bitcast_convert_type(us << 16, jnp.float32)
    gdst = gdst_ref[...]
    pdiff = gdst[:, :PW] - his[:, :PW]
    bdiff = gdst[:, PW:PW + AH] - his[:, PW:PW + AH]
    cq = cq_ref[...]
    t = jax.nn.relu(jnp.dot(pdiff, Wp1_ref[...],
                            preferred_element_type=jnp.float32) + bp1_ref[...])
    delta = jnp.dot(t, Wp2_ref[...], preferred_element_type=jnp.float32) \
        + bp2_ref[...]
    u = bdiff + jnp.dot(t, wq_ref[...], preferred_element_type=jnp.float32) \
        + cq[:, :AH]
    alpha = jnp.dot(jax.nn.relu(u), Wa2_ref[...],
                    preferred_element_type=jnp.float32) + bb2_ref[...]
    s = jnp.exp(alpha)
    sv_ref[0] = s
    sv_ref[1] = s * (xls + delta)


def _sc_scatter_body(dsti_hbm, sv_hbm, zeros_hbm, out_hbm, *slots):
    table_sh = slots[-1]
    slots = slots[:-1]
    NB = len(slots) // 5
    idxa, idxb, rowss, lsems, ssems = (slots[i * NB:(i + 1) * NB]
                                       for i in range(5))
    c = lax.axis_index("c")
    s = lax.axis_index("s")

    @pl.when(s == 0)
    def _():
        pltpu.sync_copy(zeros_hbm, table_sh)

    ept = dsti_hbm.shape[0] // NS
    nch = ept // CL
    base0 = s * ept
    plsc.subcore_barrier()

    def l_start(ch, b):
        base = base0 + pl.multiple_of(ch * CL, 8)
        pltpu.async_copy(dsti_hbm.at[pl.ds(base, CS)], idxa[b], lsems[b])
        pltpu.async_copy(dsti_hbm.at[pl.ds(base + CS, CS)], idxb[b], lsems[b])
        pltpu.async_copy(sv_hbm.at[c, pl.ds(base, CL)], rowss[b], lsems[b])

    def l_wait(b):
        pltpu.make_async_copy(dsti_hbm.at[pl.ds(0, CS)], idxa[b],
                              lsems[b]).wait()
        pltpu.make_async_copy(dsti_hbm.at[pl.ds(0, CS)], idxb[b],
                              lsems[b]).wait()
        pltpu.make_async_copy(sv_hbm.at[0, pl.ds(0, CL)], rowss[b],
                              lsems[b]).wait()

    def s_start(b):
        pltpu.async_copy(rowss[b].at[pl.ds(0, CS)], table_sh.at[idxa[b]],
                         ssems[b], add=True)
        pltpu.async_copy(rowss[b].at[pl.ds(CS, CS)], table_sh.at[idxb[b]],
                         ssems[b], add=True)

    def s_wait(b):
        pltpu.make_async_copy(rowss[b], table_sh.at[pl.ds(0, CL)],
                              ssems[b]).wait()

    for b in range(NB):
        l_start(b, b)

    def body(k, carry):
        ch = k * NB
        for b in range(NB):
            l_wait(b)
            s_start(b)
        for b in range(NB):
            s_wait(b)

            @pl.when(ch + NB + b < nch - 1)
            def _(b=b, nc=ch + NB + b):
                l_start(nc, b)
        return carry

    lax.fori_loop(0, (nch - 1) // NB, body, 0)
    l_start(nch - 1, 0)
    l_wait(0)
    s_start(0)
    s_wait(0)
    plsc.subcore_barrier()

    rpt = 624
    pltpu.sync_copy(table_sh.at[pl.ds(s * rpt, rpt)],
                    out_hbm.at[c, pl.ds(s * rpt, rpt)])

    @pl.when(s == 0)
    def _():
        pltpu.sync_copy(table_sh.at[pl.ds(NS * rpt, N - NS * rpt)],
                        out_hbm.at[c, pl.ds(NS * rpt, N - NS * rpt)])


def _final_kernel(pd0_ref, pn0_ref, pd1_ref, pn1_ref, s0_ref, v0_ref,
                  W_out_ref, b_out_ref, out_ref):
    denom = pd0_ref[0] + pd1_ref[0] + s0_ref[...]
    num = pn0_ref[0] + pn1_ref[0] + v0_ref[...]
    y = num / (denom + 1e-16)
    out_ref[...] = jax.nn.relu(
        jnp.dot(y, W_out_ref[...], preferred_element_type=jnp.float32)
        + b_out_ref[...])


def kernel(x, pos, edge_index, W_in, b_in, W_lin, W_src, W_dst, Wp1, bp1,
           Wp2, bp2, Wa1, ba1, Wa2, bb2, W_out, b_out):
    f32 = jnp.float32
    pos16 = jnp.pad(pos, ((0, 0), (0, PW - 3)))
    Wp1_16 = jnp.pad(Wp1, ((0, PW - 3), (0, 0)))
    src = edge_index[0].astype(jnp.int32)
    dst = edge_index[1].astype(jnp.int32)
    b_in2 = b_in.reshape(1, D)
    bp1_2 = bp1.reshape(1, AH)
    bp2_2 = bp2.reshape(1, D)
    ba1_2 = ba1.reshape(1, AH)
    bb2_2 = bb2.reshape(1, D)
    b_out2 = b_out.reshape(1, D)

    nsteps = N // RB
    full = lambda shp: pl.BlockSpec(shp, lambda i: tuple(0 for _ in shp))
    rows = lambda w: pl.BlockSpec((RB, w), lambda i: (i, 0))

    tsrc, tdst, s0, v0, wq, cq = pl.pallas_call(
        _node_kernel,
        grid=(nsteps,),
        in_specs=[rows(D), rows(PW), full((D, D)), full((1, D)),
                  full((D, D)), full((D, D)), full((D, D)), full((1, AH)),
                  full((AH, D)), full((1, D)), full((D, AH)), full((1, AH)),
                  full((AH, D)), full((1, D))],
        out_specs=[rows(DS), rows(DD), rows(D), rows(D),
                   full((AH, AH)), full((1, AH + D))],
        out_shape=[jax.ShapeDtypeStruct((N, DS), f32),
                   jax.ShapeDtypeStruct((N, DD), f32),
                   jax.ShapeDtypeStruct((N, D), f32),
                   jax.ShapeDtypeStruct((N, D), f32),
                   jax.ShapeDtypeStruct((AH, AH), f32),
                   jax.ShapeDtypeStruct((1, AH + D), f32)],
    )(x, pos16, W_in, b_in2, W_lin, W_src, W_dst, bp1_2, Wp2, bp2_2,
      Wa1, ba1_2, Wa2, bb2_2)

    mesh = plsc.VectorSubcoreMesh(core_axis_name="c", subcore_axis_name="s")
    H = E // NH
    zeros = jnp.zeros((N, D), f32)
    erows = lambda w: pl.BlockSpec((EB, w), lambda i: (i, 0))

    def gather_half(src_h, dst_h):
        return pl.kernel(
            _sc_gather_body,
            out_type=[jax.ShapeDtypeStruct((H, DS), f32),
                      jax.ShapeDtypeStruct((H, DD), f32)],
            mesh=mesh,
            scratch_types=([pltpu.VMEM((H // NW,), jnp.int32)] * 2
                           + [pltpu.VMEM((CG, DS), f32)] * NB
                           + [pltpu.VMEM((CG, DD), f32)] * NB
                           + [pltpu.SemaphoreType.DMA] * (2 * NB)),
        )(src_h, dst_h, tsrc, tdst)

    def edge_half(gsrc, gdst):
        return pl.pallas_call(
            _edge_kernel,
            grid=(H // EB,),
            in_specs=[erows(DS), erows(DD), full((1, AH)), full((PW, AH)),
                      full((AH, D)), full((1, D)), full((AH, AH)),
                      full((1, AH + D)), full((AH, D)), full((1, D))],
            out_specs=[pl.BlockSpec((2, EB, D), lambda i: (0, i, 0))],
            out_shape=[jax.ShapeDtypeStruct((2, H, D), f32)],
        )(gsrc, gdst, bp1_2, Wp1_16, Wp2, bp2_2, wq, cq, Wa2, bb2_2)[0]

    def scatter_half(dst_h, sv):
        return pl.kernel(
            _sc_scatter_body,
            out_type=jax.ShapeDtypeStruct((2, N, D), f32),
            mesh=mesh,
            scratch_types=([pltpu.VMEM((CS,), jnp.int32)] * (2 * NB)
                           + [pltpu.VMEM((CL, D), f32)] * NB
                           + [pltpu.SemaphoreType.DMA] * (2 * NB)
                           + [pltpu.VMEM_SHARED((N, D), f32)]),
        )(dst_h, sv, zeros)

    srcs = [lax.slice(src, (h * H,), ((h + 1) * H,)) for h in range(NH)]
    dsts = [lax.slice(dst, (h * H,), ((h + 1) * H,)) for h in range(NH)]
    gs = [gather_half(srcs[h], dsts[h]) for h in range(NH)]
    svs = [edge_half(*gs[h]) for h in range(NH)]
    parts = [scatter_half(dsts[h], svs[h]) for h in range(NH)]

    out = pl.pallas_call(
        _final_kernel,
        grid=(nsteps,),
        in_specs=[pl.BlockSpec((1, RB, D), lambda i: (0, i, 0)),
                  pl.BlockSpec((1, RB, D), lambda i: (1, i, 0)),
                  pl.BlockSpec((1, RB, D), lambda i: (0, i, 0)),
                  pl.BlockSpec((1, RB, D), lambda i: (1, i, 0)),
                  rows(D), rows(D), full((D, D)), full((1, D))],
        out_specs=[rows(D)],
        out_shape=[jax.ShapeDtypeStruct((N, D), f32)],
    )(parts[0], parts[0], parts[1], parts[1], s0, v0, W_out, b_out2)[0]
    return out
